# Initial kernel scaffold; baseline (speedup 1.0000x reference)
import jax
import jax.numpy as jnp
from jax.experimental import pallas as pl


def kernel(x, edge_index, batch, W1l, b1, W1r, W2l, b2, W2r, W3l, b3, W3r):
    raise NotImplementedError("write your pallas kernel here")


# speedup vs baseline: 6.2862x; 6.2862x over previous
import functools

import jax
import jax.numpy as jnp
from jax import lax
from jax.experimental import pallas as pl
from jax.experimental.pallas import tpu as pltpu
from jax.experimental.pallas import tpu_sc as plsc

_N = 10000
_E = 320000
_D = 128
_DOUT = 64
_G = 128
_NC = 2
_NS = 16
_CH = 128
_EPT = _E // (_NC * _NS)
_CPT = _EPT // _CH
_TAIL = _EPT - _CPT * _CH
_SLAB = 640
_LSLAB = _N - 15 * _SLAB
_ZR = 80

_f32 = jnp.float32


def _mesh():
    return plsc.VectorSubcoreMesh(core_axis_name="c", subcore_axis_name="s")


def _make_sc_agg(D):
    out_t = jax.ShapeDtypeStruct((_NC, _N, D), _f32)
    scratch = [
        pltpu.VMEM((_CH,), jnp.int32),
        pltpu.VMEM((_CH,), jnp.int32),
        pltpu.VMEM((_TAIL,), jnp.int32),
        pltpu.VMEM((_TAIL,), jnp.int32),
        pltpu.VMEM((_CH, D), _f32),
        pltpu.VMEM((_ZR, D), _f32),
        pltpu.VMEM_SHARED((_N, D), _f32),
    ]

    def body(table, src, dst, out, srcb, dstb, srct, dstt, gbuf, zbuf, acc):
        cid = lax.axis_index("c")
        sid = lax.axis_index("s")
        zv = jnp.zeros((16,), _f32)

        @pl.loop(0, _ZR)
        def _zero(r):
            for k in range(D // 16):
                zbuf[r, pl.ds(16 * k, 16)] = zv

        nb = sid * _SLAB

        @pl.when(sid < _NS - 1)
        def _z_main():
            for b in range(_SLAB // _ZR):
                pltpu.sync_copy(zbuf, acc.at[pl.ds(nb + b * _ZR, _ZR)])

        @pl.when(sid == _NS - 1)
        def _z_last():
            for b in range(_LSLAB // _ZR):
                pltpu.sync_copy(zbuf, acc.at[pl.ds(15 * _SLAB + b * _ZR, _ZR)])

        plsc.subcore_barrier()

        e0 = (cid * _NS + sid) * _EPT

        @pl.loop(0, _CPT)
        def _main(i):
            e = e0 + i * _CH
            pltpu.sync_copy(src.at[pl.ds(e, _CH)], srcb)
            pltpu.sync_copy(dst.at[pl.ds(e, _CH)], dstb)
            pltpu.sync_copy(table.at[srcb], gbuf)
            pltpu.sync_copy(gbuf, acc.at[dstb], add=True)

        et = e0 + _CPT * _CH
        pltpu.sync_copy(src.at[pl.ds(et, _TAIL)], srct)
        pltpu.sync_copy(dst.at[pl.ds(et, _TAIL)], dstt)
        pltpu.sync_copy(table.at[srct], gbuf.at[pl.ds(0, _TAIL)])
        pltpu.sync_copy(gbuf.at[pl.ds(0, _TAIL)], acc.at[dstt], add=True)

        plsc.subcore_barrier()

        @pl.when(sid < _NS - 1)
        def _rb_main():
            pltpu.sync_copy(acc.at[pl.ds(nb, _SLAB)],
                            out.at[cid, pl.ds(nb, _SLAB)])

        @pl.when(sid == _NS - 1)
        def _rb_last():
            pltpu.sync_copy(acc.at[pl.ds(15 * _SLAB, _LSLAB)],
                            out.at[cid, pl.ds(15 * _SLAB, _LSLAB)])

    return pl.kernel(body, out_type=out_t, mesh=_mesh(),
                     scratch_types=scratch,
                     compiler_params=pltpu.CompilerParams(
                         use_tc_tiling_on_sc=False))


def _make_sc_cnt():
    out_t = jax.ShapeDtypeStruct((_NC, _N, 16), _f32)
    scratch = [
        pltpu.VMEM((_CH,), jnp.int32),
        pltpu.VMEM((_TAIL,), jnp.int32),
        pltpu.VMEM((_CH, 16), _f32),
        pltpu.VMEM((_ZR, 16), _f32),
        pltpu.VMEM_SHARED((_N, 16), _f32),
    ]

    def body(dst, out, dstb, dstt, ones, z16, cacc):
        cid = lax.axis_index("c")
        sid = lax.axis_index("s")
        zv = jnp.zeros((16,), _f32)
        ov = jnp.ones((16,), _f32)

        @pl.loop(0, _CH)
        def _fill(r):
            ones[r, pl.ds(0, 16)] = ov

        @pl.loop(0, _ZR)
        def _zero(r):
            z16[r, pl.ds(0, 16)] = zv

        nb = sid * _SLAB

        @pl.when(sid < _NS - 1)
        def _z_main():
            for b in range(_SLAB // _ZR):
                pltpu.sync_copy(z16, cacc.at[pl.ds(nb + b * _ZR, _ZR)])

        @pl.when(sid == _NS - 1)
        def _z_last():
            for b in range(_LSLAB // _ZR):
                pltpu.sync_copy(z16, cacc.at[pl.ds(15 * _SLAB + b * _ZR, _ZR)])

        plsc.subcore_barrier()

        e0 = (cid * _NS + sid) * _EPT

        @pl.loop(0, _CPT)
        def _main(i):
            pltpu.sync_copy(dst.at[pl.ds(e0 + i * _CH, _CH)], dstb)
            pltpu.sync_copy(ones, cacc.at[dstb], add=True)

        et = e0 + _CPT * _CH
        pltpu.sync_copy(dst.at[pl.ds(et, _TAIL)], dstt)
        pltpu.sync_copy(ones.at[pl.ds(0, _TAIL)], cacc.at[dstt], add=True)

        plsc.subcore_barrier()

        @pl.when(sid < _NS - 1)
        def _rb_main():
            pltpu.sync_copy(cacc.at[pl.ds(nb, _SLAB)],
                            out.at[cid, pl.ds(nb, _SLAB)])

        @pl.when(sid == _NS - 1)
        def _rb_last():
            pltpu.sync_copy(cacc.at[pl.ds(15 * _SLAB, _LSLAB)],
                            out.at[cid, pl.ds(15 * _SLAB, _LSLAB)])

    return pl.kernel(body, out_type=out_t, mesh=_mesh(),
                     scratch_types=scratch,
                     compiler_params=pltpu.CompilerParams(
                         use_tc_tiling_on_sc=False))


_agg128 = _make_sc_agg(_D)
_agg64 = _make_sc_agg(_DOUT)
_cntk = _make_sc_cnt()


_PC = 16
_NCHK = _N // _PC
_W = _NC * _NS
_ITER = -(-_NCHK // _W)


def _make_sc_pool():
    outs = (jax.ShapeDtypeStruct((_NC, _G, _DOUT), _f32),
            jax.ShapeDtypeStruct((_NC, _G, _D), _f32),
            jax.ShapeDtypeStruct((_NC, _G, 16), _f32))
    scratch = [
        pltpu.VMEM((_PC, _DOUT), _f32),
        pltpu.VMEM((_PC, _DOUT), _f32),
        pltpu.VMEM((_PC, 16), _f32),
        pltpu.VMEM((_PC, 16), _f32),
        pltpu.VMEM((_PC, _D), _f32),
        pltpu.VMEM((_PC,), jnp.int32),
        pltpu.VMEM((_PC, _DOUT), _f32),
        pltpu.VMEM((_PC, 16), _f32),
        pltpu.VMEM((_PC, _D), _f32),
        pltpu.VMEM_SHARED((_G, _DOUT), _f32),
        pltpu.VMEM_SHARED((_G, _D), _f32),
        pltpu.VMEM_SHARED((_G, 16), _f32),
    ]

    def body(agg3, cnt, h2, batch, pz_out, ph_out, gc_out,
             a0, a1, c0, c1, hb, bb, zb, ones, zz, pzacc, phacc, gcacc):
        cid = lax.axis_index("c")
        sid = lax.axis_index("s")
        w = cid * _NS + sid
        zv = jnp.zeros((16,), _f32)
        ov = jnp.ones((16,), _f32)
        for r in range(_PC):
            for k in range(_D // 16):
                zz[r, pl.ds(16 * k, 16)] = zv
            for k in range(_DOUT // 16):
                zb[r, pl.ds(16 * k, 16)] = zv
            c0[r, pl.ds(0, 16)] = zv
            ones[r, pl.ds(0, 16)] = ov

        @pl.when(sid < _G // _PC)
        def _init():
            rb = sid * _PC
            pltpu.sync_copy(zb, pzacc.at[pl.ds(rb, _PC)])
            pltpu.sync_copy(zz, phacc.at[pl.ds(rb, _PC)])
            pltpu.sync_copy(c0, gcacc.at[pl.ds(rb, _PC)])

        plsc.subcore_barrier()

        @pl.loop(0, _ITER)
        def _main(it):
            ch = w + _W * it

            @pl.when(ch < _NCHK)
            def _do():
                o = ch * _PC
                pltpu.sync_copy(agg3.at[0, pl.ds(o, _PC)], a0)
                pltpu.sync_copy(agg3.at[1, pl.ds(o, _PC)], a1)
                pltpu.sync_copy(cnt.at[0, pl.ds(o, _PC)], c0)
                pltpu.sync_copy(cnt.at[1, pl.ds(o, _PC)], c1)
                pltpu.sync_copy(h2.at[pl.ds(o, _PC)], hb)
                pltpu.sync_copy(batch.at[pl.ds(o, _PC)], bb)
                for r in range(_PC):
                    cv = jnp.maximum(
                        c0[r, pl.ds(0, 16)] + c1[r, pl.ds(0, 16)], 1.0)
                    for k in range(_DOUT // 16):
                        s = pl.ds(16 * k, 16)
                        zb[r, s] = (a0[r, s] + a1[r, s]) / cv
                pltpu.sync_copy(zb, pzacc.at[bb], add=True)
                pltpu.sync_copy(hb, phacc.at[bb], add=True)
                pltpu.sync_copy(ones, gcacc.at[bb], add=True)

        plsc.subcore_barrier()

        @pl.when(sid == 0)
        def _o0():
            pltpu.sync_copy(pzacc, pz_out.at[cid])

        @pl.when(sid == 1)
        def _o1():
            pltpu.sync_copy(phacc, ph_out.at[cid])

        @pl.when(sid == 2)
        def _o2():
            pltpu.sync_copy(gcacc, gc_out.at[cid])

    return pl.kernel(body, out_type=outs, mesh=_mesh(),
                     scratch_types=scratch,
                     compiler_params=pltpu.CompilerParams(
                         use_tc_tiling_on_sc=False))


_sc_pool = _make_sc_pool()


_R = 1000


def _dense_body(agg, cnt, xin, wl, b, wr, *rest, relu, with_y):
    if with_y:
        w3, h_out, y_out = rest
    else:
        (h_out,) = rest
    a = agg[0] + agg[1]
    c = cnt[0, :, 0:1] + cnt[1, :, 0:1]
    mean = a * (1.0 / jnp.maximum(c, 1.0))
    h = lax.dot_general(mean, wl[...], (((1,), (1,)), ((), ())),
                        preferred_element_type=_f32)
    h = h + b[...] + lax.dot_general(xin[...], wr[...], (((1,), (1,)), ((), ())),
                                     preferred_element_type=_f32)
    if relu:
        h = jnp.maximum(h, 0.0)
    h_out[...] = h
    if with_y:
        y_out[...] = lax.dot_general(h, w3[...], (((1,), (1,)), ((), ())),
                                     preferred_element_type=_f32)


def _make_dense(relu, with_y):
    in_specs = [
        pl.BlockSpec((_NC, _R, _D), lambda i: (0, i, 0)),
        pl.BlockSpec((_NC, _R, 16), lambda i: (0, i, 0)),
        pl.BlockSpec((_R, _D), lambda i: (i, 0)),
        pl.BlockSpec((_D, _D), lambda i: (0, 0)),
        pl.BlockSpec((1, _D), lambda i: (0, 0)),
        pl.BlockSpec((_D, _D), lambda i: (0, 0)),
    ]
    out_shape = [jax.ShapeDtypeStruct((_N, _D), _f32)]
    out_specs = [pl.BlockSpec((_R, _D), lambda i: (i, 0))]
    if with_y:
        in_specs.append(pl.BlockSpec((_DOUT, _D), lambda i: (0, 0)))
        out_shape.append(jax.ShapeDtypeStruct((_N, _DOUT), _f32))
        out_specs.append(pl.BlockSpec((_R, _DOUT), lambda i: (i, 0)))
    return pl.pallas_call(
        functools.partial(_dense_body, relu=relu, with_y=with_y),
        grid=(_N // _R,),
        in_specs=in_specs,
        out_specs=out_specs if with_y else out_specs[0],
        out_shape=out_shape if with_y else out_shape[0],
    )


_dense1 = _make_dense(True, False)
_dense2 = _make_dense(True, True)


def _final_body(pz, ph, gc, wr, b, out):
    z = pz[0] + pz[1]
    h = ph[0] + ph[1]
    g = gc[0, :, 0:1] + gc[1, :, 0:1]
    s = z + g * b[...] + lax.dot_general(h, wr[...], (((1,), (1,)), ((), ())),
                                         preferred_element_type=_f32)
    out[...] = s / jnp.maximum(g, 1.0)


_final = pl.pallas_call(
    _final_body,
    out_shape=jax.ShapeDtypeStruct((_G, _DOUT), _f32),
)


def kernel(x, edge_index, batch, W1l, b1, W1r, W2l, b2, W2r, W3l, b3, W3r):
    src = edge_index[0]
    dst = edge_index[1]
    cnt = _cntk(dst)
    agg1 = _agg128(x, src, dst)
    h1 = _dense1(agg1, cnt, x, W1l, b1.reshape(1, -1), W1r)
    agg2 = _agg128(h1, src, dst)
    h2, y3l = _dense2(agg2, cnt, h1, W2l, b2.reshape(1, -1), W2r, W3l)
    agg3 = _agg64(y3l, src, dst)
    pz, ph, gc = _sc_pool(agg3, cnt, h2, batch)
    return _final(pz, ph, gc, W3r, b3.reshape(1, -1))


# speedup vs baseline: 10.8824x; 1.7312x over previous
import functools

import jax
import jax.numpy as jnp
from jax import lax
from jax.experimental import pallas as pl
from jax.experimental.pallas import tpu as pltpu
from jax.experimental.pallas import tpu_sc as plsc

_N = 10000
_E = 320000
_D = 128
_DOUT = 64
_G = 128
_NC = 2
_NS = 16
_CH = 64
_EPT = _E // (_NC * _NS)
_CPT = _EPT // _CH
_TAIL = _EPT - _CPT * _CH
_SLAB = 640
_LSLAB = _N - 15 * _SLAB
_ZR = 8

_f32 = jnp.float32


def _mesh():
    return plsc.VectorSubcoreMesh(core_axis_name="c", subcore_axis_name="s")


def _make_sc_agg(D):
    out_t = jax.ShapeDtypeStruct((_NC, _N, D), _f32)
    scratch = [
        pltpu.VMEM((_EPT,), jnp.int32),
        pltpu.VMEM((_EPT,), jnp.int32),
        pltpu.VMEM((_CH, D), _f32),
        pltpu.VMEM((_CH, D), _f32),
        pltpu.VMEM((_ZR, D), _f32),
        pltpu.VMEM_SHARED((_N, D), _f32),
        pltpu.SemaphoreType.DMA,
        pltpu.SemaphoreType.DMA,
        pltpu.SemaphoreType.DMA,
        pltpu.SemaphoreType.DMA,
        pltpu.SemaphoreType.DMA,
    ]

    def body(table, src, dst, out, srcall, dstall, gb0, gb1, zbuf, acc,
             gs0, gs1, ss0, ss1, zs):
        cid = lax.axis_index("c")
        sid = lax.axis_index("s")
        zv = jnp.zeros((16,), _f32)

        e0 = (cid * _NS + sid) * _EPT
        pltpu.async_copy(src.at[pl.ds(e0, _EPT)], srcall, gs0)
        pltpu.async_copy(dst.at[pl.ds(e0, _EPT)], dstall, gs1)

        @pl.loop(0, _ZR)
        def _zero(r):
            for k in range(D // 16):
                zbuf[r, pl.ds(16 * k, 16)] = zv

        nb = sid * _SLAB

        @pl.when(sid < _NS - 1)
        def _z_main():
            for b in range(_SLAB // _ZR):
                pltpu.async_copy(zbuf, acc.at[pl.ds(nb + b * _ZR, _ZR)], zs)
            for b in range(_SLAB // _ZR):
                pltpu.make_async_copy(zbuf, acc.at[pl.ds(nb, _ZR)], zs).wait()

        @pl.when(sid == _NS - 1)
        def _z_last():
            for b in range(_LSLAB // _ZR):
                pltpu.async_copy(
                    zbuf, acc.at[pl.ds(15 * _SLAB + b * _ZR, _ZR)], zs)
            for b in range(_LSLAB // _ZR):
                pltpu.make_async_copy(zbuf, acc.at[pl.ds(nb, _ZR)], zs).wait()

        pltpu.make_async_copy(src.at[pl.ds(e0, _EPT)], srcall, gs0).wait()
        pltpu.make_async_copy(dst.at[pl.ds(e0, _EPT)], dstall, gs1).wait()
        plsc.subcore_barrier()

        def start_gather(j, buf, sem):
            pltpu.async_copy(table.at[srcall.at[pl.ds(j * _CH, _CH)]],
                             buf, sem)

        def wait_gather(buf, sem):
            pltpu.make_async_copy(
                table.at[srcall.at[pl.ds(0, _CH)]], buf, sem).wait()

        def start_scatter(j, buf, sem):
            pltpu.async_copy(buf, acc.at[dstall.at[pl.ds(j * _CH, _CH)]],
                             sem, add=True)

        def wait_scatter(buf, sem):
            pltpu.make_async_copy(
                buf, acc.at[dstall.at[pl.ds(0, _CH)]], sem).wait()

        start_gather(0, gb0, gs0)

        @pl.loop(0, _CPT // 2)
        def _pair(p):
            @pl.when(p > 0)
            def _w1():
                wait_scatter(gb1, ss1)
            start_gather(2 * p + 1, gb1, gs1)
            wait_gather(gb0, gs0)
            start_scatter(2 * p, gb0, ss0)
            wait_scatter(gb0, ss0)

            @pl.when(p < _CPT // 2 - 1)
            def _g2():
                start_gather(2 * p + 2, gb0, gs0)
            wait_gather(gb1, gs1)
            start_scatter(2 * p + 1, gb1, ss1)

        wait_scatter(gb1, ss1)
        et = _CPT * _CH
        pltpu.sync_copy(table.at[srcall.at[pl.ds(et, _TAIL)]],
                        gb0.at[pl.ds(0, _TAIL)])
        pltpu.sync_copy(gb0.at[pl.ds(0, _TAIL)],
                        acc.at[dstall.at[pl.ds(et, _TAIL)]], add=True)

        plsc.subcore_barrier()

        @pl.when(sid < _NS - 1)
        def _rb_main():
            pltpu.sync_copy(acc.at[pl.ds(nb, _SLAB)],
                            out.at[cid, pl.ds(nb, _SLAB)])

        @pl.when(sid == _NS - 1)
        def _rb_last():
            pltpu.sync_copy(acc.at[pl.ds(15 * _SLAB, _LSLAB)],
                            out.at[cid, pl.ds(15 * _SLAB, _LSLAB)])

    return pl.kernel(body, out_type=out_t, mesh=_mesh(),
                     scratch_types=scratch,
                     compiler_params=pltpu.CompilerParams(
                         use_tc_tiling_on_sc=False))


def _make_sc_cnt():
    out_t = jax.ShapeDtypeStruct((_NC, _N, 16), _f32)
    scratch = [
        pltpu.VMEM((_EPT,), jnp.int32),
        pltpu.VMEM((_CH, 16), _f32),
        pltpu.VMEM((_ZR, 16), _f32),
        pltpu.VMEM_SHARED((_N, 16), _f32),
        pltpu.SemaphoreType.DMA,
        pltpu.SemaphoreType.DMA,
        pltpu.SemaphoreType.DMA,
    ]

    def body(dst, out, dstall, ones, z16, cacc, isem, zs, ss):
        cid = lax.axis_index("c")
        sid = lax.axis_index("s")
        zv = jnp.zeros((16,), _f32)
        ov = jnp.ones((16,), _f32)

        e0 = (cid * _NS + sid) * _EPT
        pltpu.async_copy(dst.at[pl.ds(e0, _EPT)], dstall, isem)

        @pl.loop(0, _CH)
        def _fill(r):
            ones[r, pl.ds(0, 16)] = ov

        @pl.loop(0, _ZR)
        def _zero(r):
            z16[r, pl.ds(0, 16)] = zv

        nb = sid * _SLAB

        @pl.when(sid < _NS - 1)
        def _z_main():
            for b in range(_SLAB // _ZR):
                pltpu.async_copy(z16, cacc.at[pl.ds(nb + b * _ZR, _ZR)], zs)
            for b in range(_SLAB // _ZR):
                pltpu.make_async_copy(z16, cacc.at[pl.ds(nb, _ZR)], zs).wait()

        @pl.when(sid == _NS - 1)
        def _z_last():
            for b in range(_LSLAB // _ZR):
                pltpu.async_copy(
                    z16, cacc.at[pl.ds(15 * _SLAB + b * _ZR, _ZR)], zs)
            for b in range(_LSLAB // _ZR):
                pltpu.make_async_copy(z16, cacc.at[pl.ds(nb, _ZR)], zs).wait()

        pltpu.make_async_copy(dst.at[pl.ds(e0, _EPT)], dstall, isem).wait()
        plsc.subcore_barrier()

        @pl.loop(0, _CPT)
        def _main(i):
            pltpu.async_copy(ones, cacc.at[dstall.at[pl.ds(i * _CH, _CH)]],
                             ss, add=True)

        et = _CPT * _CH
        pltpu.async_copy(ones.at[pl.ds(0, _TAIL)],
                         cacc.at[dstall.at[pl.ds(et, _TAIL)]], ss, add=True)

        @pl.loop(0, _CPT)
        def _drain(i):
            pltpu.make_async_copy(
                ones, cacc.at[dstall.at[pl.ds(0, _CH)]], ss).wait()

        pltpu.make_async_copy(
            ones.at[pl.ds(0, _TAIL)],
            cacc.at[dstall.at[pl.ds(0, _TAIL)]], ss).wait()

        plsc.subcore_barrier()

        @pl.when(sid < _NS - 1)
        def _rb_main():
            pltpu.sync_copy(cacc.at[pl.ds(nb, _SLAB)],
                            out.at[cid, pl.ds(nb, _SLAB)])

        @pl.when(sid == _NS - 1)
        def _rb_last():
            pltpu.sync_copy(cacc.at[pl.ds(15 * _SLAB, _LSLAB)],
                            out.at[cid, pl.ds(15 * _SLAB, _LSLAB)])

    return pl.kernel(body, out_type=out_t, mesh=_mesh(),
                     scratch_types=scratch,
                     compiler_params=pltpu.CompilerParams(
                         use_tc_tiling_on_sc=False))


_agg128 = _make_sc_agg(_D)
_agg64 = _make_sc_agg(_DOUT)
_cntk = _make_sc_cnt()


_PC = 16
_NCHK = _N // _PC
_W = _NC * _NS
_ITER = -(-_NCHK // _W)


def _make_sc_pool():
    outs = (jax.ShapeDtypeStruct((_NC, _G, _DOUT), _f32),
            jax.ShapeDtypeStruct((_NC, _G, _D), _f32),
            jax.ShapeDtypeStruct((_NC, _G, 16), _f32))
    scratch = [
        pltpu.VMEM((_PC, _DOUT), _f32),
        pltpu.VMEM((_PC, _DOUT), _f32),
        pltpu.VMEM((_PC, 16), _f32),
        pltpu.VMEM((_PC, 16), _f32),
        pltpu.VMEM((_PC, _D), _f32),
        pltpu.VMEM((_PC,), jnp.int32),
        pltpu.VMEM((_PC, _DOUT), _f32),
        pltpu.VMEM((_PC, 16), _f32),
        pltpu.VMEM((_PC, _D), _f32),
        pltpu.VMEM_SHARED((_G, _DOUT), _f32),
        pltpu.VMEM_SHARED((_G, _D), _f32),
        pltpu.VMEM_SHARED((_G, 16), _f32),
    ]

    def body(agg3, cnt, h2, batch, pz_out, ph_out, gc_out,
             a0, a1, c0, c1, hb, bb, zb, ones, zz, pzacc, phacc, gcacc):
        cid = lax.axis_index("c")
        sid = lax.axis_index("s")
        w = cid * _NS + sid
        zv = jnp.zeros((16,), _f32)
        ov = jnp.ones((16,), _f32)
        for r in range(_PC):
            for k in range(_D // 16):
                zz[r, pl.ds(16 * k, 16)] = zv
            for k in range(_DOUT // 16):
                zb[r, pl.ds(16 * k, 16)] = zv
            c0[r, pl.ds(0, 16)] = zv
            ones[r, pl.ds(0, 16)] = ov

        @pl.when(sid < _G // _PC)
        def _init():
            rb = sid * _PC
            pltpu.sync_copy(zb, pzacc.at[pl.ds(rb, _PC)])
            pltpu.sync_copy(zz, phacc.at[pl.ds(rb, _PC)])
            pltpu.sync_copy(c0, gcacc.at[pl.ds(rb, _PC)])

        plsc.subcore_barrier()

        @pl.loop(0, _ITER)
        def _main(it):
            ch = w + _W * it

            @pl.when(ch < _NCHK)
            def _do():
                o = ch * _PC
                pltpu.sync_copy(agg3.at[0, pl.ds(o, _PC)], a0)
                pltpu.sync_copy(agg3.at[1, pl.ds(o, _PC)], a1)
                pltpu.sync_copy(cnt.at[0, pl.ds(o, _PC)], c0)
                pltpu.sync_copy(cnt.at[1, pl.ds(o, _PC)], c1)
                pltpu.sync_copy(h2.at[pl.ds(o, _PC)], hb)
                pltpu.sync_copy(batch.at[pl.ds(o, _PC)], bb)
                for r in range(_PC):
                    cv = jnp.maximum(
                        c0[r, pl.ds(0, 16)] + c1[r, pl.ds(0, 16)], 1.0)
                    for k in range(_DOUT // 16):
                        s = pl.ds(16 * k, 16)
                        zb[r, s] = (a0[r, s] + a1[r, s]) / cv
                pltpu.sync_copy(zb, pzacc.at[bb], add=True)
                pltpu.sync_copy(hb, phacc.at[bb], add=True)
                pltpu.sync_copy(ones, gcacc.at[bb], add=True)

        plsc.subcore_barrier()

        @pl.when(sid == 0)
        def _o0():
            pltpu.sync_copy(pzacc, pz_out.at[cid])

        @pl.when(sid == 1)
        def _o1():
            pltpu.sync_copy(phacc, ph_out.at[cid])

        @pl.when(sid == 2)
        def _o2():
            pltpu.sync_copy(gcacc, gc_out.at[cid])

    return pl.kernel(body, out_type=outs, mesh=_mesh(),
                     scratch_types=scratch,
                     compiler_params=pltpu.CompilerParams(
                         use_tc_tiling_on_sc=False))


_sc_pool = _make_sc_pool()


_R = 1000


def _dense_body(agg, cnt, xin, wl, b, wr, *rest, relu, with_y):
    if with_y:
        w3, h_out, y_out = rest
    else:
        (h_out,) = rest
    a = agg[0] + agg[1]
    c = cnt[0, :, 0:1] + cnt[1, :, 0:1]
    mean = a * (1.0 / jnp.maximum(c, 1.0))
    h = lax.dot_general(mean, wl[...], (((1,), (1,)), ((), ())),
                        preferred_element_type=_f32)
    h = h + b[...] + lax.dot_general(xin[...], wr[...], (((1,), (1,)), ((), ())),
                                     preferred_element_type=_f32)
    if relu:
        h = jnp.maximum(h, 0.0)
    h_out[...] = h
    if with_y:
        y_out[...] = lax.dot_general(h, w3[...], (((1,), (1,)), ((), ())),
                                     preferred_element_type=_f32)


def _make_dense(relu, with_y):
    in_specs = [
        pl.BlockSpec((_NC, _R, _D), lambda i: (0, i, 0)),
        pl.BlockSpec((_NC, _R, 16), lambda i: (0, i, 0)),
        pl.BlockSpec((_R, _D), lambda i: (i, 0)),
        pl.BlockSpec((_D, _D), lambda i: (0, 0)),
        pl.BlockSpec((1, _D), lambda i: (0, 0)),
        pl.BlockSpec((_D, _D), lambda i: (0, 0)),
    ]
    out_shape = [jax.ShapeDtypeStruct((_N, _D), _f32)]
    out_specs = [pl.BlockSpec((_R, _D), lambda i: (i, 0))]
    if with_y:
        in_specs.append(pl.BlockSpec((_DOUT, _D), lambda i: (0, 0)))
        out_shape.append(jax.ShapeDtypeStruct((_N, _DOUT), _f32))
        out_specs.append(pl.BlockSpec((_R, _DOUT), lambda i: (i, 0)))
    return pl.pallas_call(
        functools.partial(_dense_body, relu=relu, with_y=with_y),
        grid=(_N // _R,),
        in_specs=in_specs,
        out_specs=out_specs if with_y else out_specs[0],
        out_shape=out_shape if with_y else out_shape[0],
    )


_dense1 = _make_dense(True, False)
_dense2 = _make_dense(True, True)


def _final_body(pz, ph, gc, wr, b, out):
    z = pz[0] + pz[1]
    h = ph[0] + ph[1]
    g = gc[0, :, 0:1] + gc[1, :, 0:1]
    s = z + g * b[...] + lax.dot_general(h, wr[...], (((1,), (1,)), ((), ())),
                                         preferred_element_type=_f32)
    out[...] = s / jnp.maximum(g, 1.0)


_final = pl.pallas_call(
    _final_body,
    out_shape=jax.ShapeDtypeStruct((_G, _DOUT), _f32),
)


def kernel(x, edge_index, batch, W1l, b1, W1r, W2l, b2, W2r, W3l, b3, W3r):
    src = edge_index[0]
    dst = edge_index[1]
    cnt = _cntk(dst)
    agg1 = _agg128(x, src, dst)
    h1 = _dense1(agg1, cnt, x, W1l, b1.reshape(1, -1), W1r)
    agg2 = _agg128(h1, src, dst)
    h2, y3l = _dense2(agg2, cnt, h1, W2l, b2.reshape(1, -1), W2r, W3l)
    agg3 = _agg64(y3l, src, dst)
    pz, ph, gc = _sc_pool(agg3, cnt, h2, batch)
    return _final(pz, ph, gc, W3r, b3.reshape(1, -1))


# speedup vs baseline: 12.3409x; 1.1340x over previous
import functools

import jax
import jax.numpy as jnp
from jax import lax
from jax.experimental import pallas as pl
from jax.experimental.pallas import tpu as pltpu
from jax.experimental.pallas import tpu_sc as plsc

_N = 10000
_E = 320000
_D = 128
_DOUT = 64
_G = 128
_NC = 2
_NS = 16
_CH = 64
_EPT = _E // (_NC * _NS)
_CPT = _EPT // _CH
_TAIL = _EPT - _CPT * _CH
_SLAB = 640
_LSLAB = _N - 15 * _SLAB
_ZR = 8

_f32 = jnp.float32


def _mesh():
    return plsc.VectorSubcoreMesh(core_axis_name="c", subcore_axis_name="s")


def _make_sc_agg(D):
    out_t = jax.ShapeDtypeStruct((_NC, _N, D), _f32)
    scratch = [
        pltpu.VMEM((_EPT,), jnp.int32),
        pltpu.VMEM((_EPT,), jnp.int32),
        pltpu.VMEM((_CH, D), _f32),
        pltpu.VMEM((_CH, D), _f32),
        pltpu.VMEM((_ZR, D), _f32),
        pltpu.VMEM_SHARED((_N, D), _f32),
        pltpu.SemaphoreType.DMA,
        pltpu.SemaphoreType.DMA,
        pltpu.SemaphoreType.DMA,
        pltpu.SemaphoreType.DMA,
        pltpu.SemaphoreType.DMA,
    ]

    def body(table, src, dst, out, srcall, dstall, gb0, gb1, zbuf, acc,
             gs0, gs1, ss0, ss1, zs):
        cid = lax.axis_index("c")
        sid = lax.axis_index("s")
        zv = jnp.zeros((16,), _f32)

        e0 = (cid * _NS + sid) * _EPT
        pltpu.async_copy(src.at[pl.ds(e0, _EPT)], srcall, gs0)
        pltpu.async_copy(dst.at[pl.ds(e0, _EPT)], dstall, gs1)

        @pl.loop(0, _ZR)
        def _zero(r):
            for k in range(D // 16):
                zbuf[r, pl.ds(16 * k, 16)] = zv

        nb = sid * _SLAB

        @pl.when(sid < _NS - 1)
        def _z_main():
            for b in range(_SLAB // _ZR):
                pltpu.async_copy(zbuf, acc.at[pl.ds(nb + b * _ZR, _ZR)], zs)
            for b in range(_SLAB // _ZR):
                pltpu.make_async_copy(zbuf, acc.at[pl.ds(nb, _ZR)], zs).wait()

        @pl.when(sid == _NS - 1)
        def _z_last():
            for b in range(_LSLAB // _ZR):
                pltpu.async_copy(
                    zbuf, acc.at[pl.ds(15 * _SLAB + b * _ZR, _ZR)], zs)
            for b in range(_LSLAB // _ZR):
                pltpu.make_async_copy(zbuf, acc.at[pl.ds(nb, _ZR)], zs).wait()

        pltpu.make_async_copy(src.at[pl.ds(e0, _EPT)], srcall, gs0).wait()
        pltpu.make_async_copy(dst.at[pl.ds(e0, _EPT)], dstall, gs1).wait()
        plsc.subcore_barrier()

        def start_gather(j, buf, sem):
            pltpu.async_copy(table.at[srcall.at[pl.ds(j * _CH, _CH)]],
                             buf, sem)

        def wait_gather(buf, sem):
            pltpu.make_async_copy(
                table.at[srcall.at[pl.ds(0, _CH)]], buf, sem).wait()

        def start_scatter(j, buf, sem):
            pltpu.async_copy(buf, acc.at[dstall.at[pl.ds(j * _CH, _CH)]],
                             sem, add=True)

        def wait_scatter(buf, sem):
            pltpu.make_async_copy(
                buf, acc.at[dstall.at[pl.ds(0, _CH)]], sem).wait()

        start_gather(0, gb0, gs0)

        @pl.loop(0, _CPT // 2)
        def _pair(p):
            @pl.when(p > 0)
            def _w1():
                wait_scatter(gb1, ss1)
            start_gather(2 * p + 1, gb1, gs1)
            wait_gather(gb0, gs0)
            start_scatter(2 * p, gb0, ss0)
            wait_scatter(gb0, ss0)

            @pl.when(p < _CPT // 2 - 1)
            def _g2():
                start_gather(2 * p + 2, gb0, gs0)
            wait_gather(gb1, gs1)
            start_scatter(2 * p + 1, gb1, ss1)

        wait_scatter(gb1, ss1)
        et = _CPT * _CH
        pltpu.sync_copy(table.at[srcall.at[pl.ds(et, _TAIL)]],
                        gb0.at[pl.ds(0, _TAIL)])
        pltpu.sync_copy(gb0.at[pl.ds(0, _TAIL)],
                        acc.at[dstall.at[pl.ds(et, _TAIL)]], add=True)

        plsc.subcore_barrier()

        @pl.when(sid < _NS - 1)
        def _rb_main():
            pltpu.sync_copy(acc.at[pl.ds(nb, _SLAB)],
                            out.at[cid, pl.ds(nb, _SLAB)])

        @pl.when(sid == _NS - 1)
        def _rb_last():
            pltpu.sync_copy(acc.at[pl.ds(15 * _SLAB, _LSLAB)],
                            out.at[cid, pl.ds(15 * _SLAB, _LSLAB)])

    return pl.kernel(body, out_type=out_t, mesh=_mesh(),
                     scratch_types=scratch,
                     compiler_params=pltpu.CompilerParams(
                         use_tc_tiling_on_sc=False))


def _make_sc_cnt():
    out_t = jax.ShapeDtypeStruct((_NC, _N, 16), _f32)
    scratch = [
        pltpu.VMEM((_EPT,), jnp.int32),
        pltpu.VMEM((_CH, 16), _f32),
        pltpu.VMEM((_ZR, 16), _f32),
        pltpu.VMEM_SHARED((_N, 16), _f32),
        pltpu.SemaphoreType.DMA,
        pltpu.SemaphoreType.DMA,
        pltpu.SemaphoreType.DMA,
    ]

    def body(dst, out, dstall, ones, z16, cacc, isem, zs, ss):
        cid = lax.axis_index("c")
        sid = lax.axis_index("s")
        zv = jnp.zeros((16,), _f32)
        ov = jnp.ones((16,), _f32)

        e0 = (cid * _NS + sid) * _EPT
        pltpu.async_copy(dst.at[pl.ds(e0, _EPT)], dstall, isem)

        @pl.loop(0, _CH)
        def _fill(r):
            ones[r, pl.ds(0, 16)] = ov

        @pl.loop(0, _ZR)
        def _zero(r):
            z16[r, pl.ds(0, 16)] = zv

        nb = sid * _SLAB

        @pl.when(sid < _NS - 1)
        def _z_main():
            for b in range(_SLAB // _ZR):
                pltpu.async_copy(z16, cacc.at[pl.ds(nb + b * _ZR, _ZR)], zs)
            for b in range(_SLAB // _ZR):
                pltpu.make_async_copy(z16, cacc.at[pl.ds(nb, _ZR)], zs).wait()

        @pl.when(sid == _NS - 1)
        def _z_last():
            for b in range(_LSLAB // _ZR):
                pltpu.async_copy(
                    z16, cacc.at[pl.ds(15 * _SLAB + b * _ZR, _ZR)], zs)
            for b in range(_LSLAB // _ZR):
                pltpu.make_async_copy(z16, cacc.at[pl.ds(nb, _ZR)], zs).wait()

        pltpu.make_async_copy(dst.at[pl.ds(e0, _EPT)], dstall, isem).wait()
        plsc.subcore_barrier()

        @pl.loop(0, _CPT)
        def _main(i):
            pltpu.async_copy(ones, cacc.at[dstall.at[pl.ds(i * _CH, _CH)]],
                             ss, add=True)

        et = _CPT * _CH
        pltpu.async_copy(ones.at[pl.ds(0, _TAIL)],
                         cacc.at[dstall.at[pl.ds(et, _TAIL)]], ss, add=True)

        @pl.loop(0, _CPT)
        def _drain(i):
            pltpu.make_async_copy(
                ones, cacc.at[dstall.at[pl.ds(0, _CH)]], ss).wait()

        pltpu.make_async_copy(
            ones.at[pl.ds(0, _TAIL)],
            cacc.at[dstall.at[pl.ds(0, _TAIL)]], ss).wait()

        plsc.subcore_barrier()

        @pl.when(sid < _NS - 1)
        def _rb_main():
            pltpu.sync_copy(cacc.at[pl.ds(nb, _SLAB)],
                            out.at[cid, pl.ds(nb, _SLAB)])

        @pl.when(sid == _NS - 1)
        def _rb_last():
            pltpu.sync_copy(cacc.at[pl.ds(15 * _SLAB, _LSLAB)],
                            out.at[cid, pl.ds(15 * _SLAB, _LSLAB)])

    return pl.kernel(body, out_type=out_t, mesh=_mesh(),
                     scratch_types=scratch,
                     compiler_params=pltpu.CompilerParams(
                         use_tc_tiling_on_sc=False))


_agg128 = _make_sc_agg(_D)
_agg64 = _make_sc_agg(_DOUT)
_cntk = _make_sc_cnt()


_PC = 16
_NCHK = _N // _PC
_W = _NC * _NS
_ITER = -(-_NCHK // _W)


def _make_sc_pool():
    outs = (jax.ShapeDtypeStruct((_NC, _G, _DOUT), _f32),
            jax.ShapeDtypeStruct((_NC, _G, _D), _f32),
            jax.ShapeDtypeStruct((_NC, _G, 16), _f32))
    nbuf = 2
    scratch = (
        [pltpu.VMEM((_PC, _DOUT), _f32)] * nbuf +
        [pltpu.VMEM((_PC, _DOUT), _f32)] * nbuf +
        [pltpu.VMEM((_PC, 16), _f32)] * nbuf +
        [pltpu.VMEM((_PC, 16), _f32)] * nbuf +
        [pltpu.VMEM((_PC, _D), _f32)] * nbuf +
        [pltpu.VMEM((_PC,), jnp.int32)] * nbuf +
        [pltpu.VMEM((_PC, _DOUT), _f32)] * nbuf +
        [
            pltpu.VMEM((_PC, 16), _f32),
            pltpu.VMEM((_PC, _D), _f32),
            pltpu.VMEM_SHARED((_G, _DOUT), _f32),
            pltpu.VMEM_SHARED((_G, _D), _f32),
            pltpu.VMEM_SHARED((_G, 16), _f32),
        ] +
        [pltpu.SemaphoreType.DMA] * nbuf +
        [pltpu.SemaphoreType.DMA] * nbuf
    )

    def body(agg3, cnt, h2, batch, pz_out, ph_out, gc_out,
             a00, a01, a10, a11, c00, c01, c10, c11, hb0, hb1, bb0, bb1,
             zb0, zb1, ones, zz, pzacc, phacc, gcacc, ls0, ls1, ss0, ss1):
        a0 = (a00, a01); a1 = (a10, a11); c0 = (c00, c01); c1 = (c10, c11)
        hb = (hb0, hb1); bb = (bb0, bb1); zb = (zb0, zb1)
        ls = (ls0, ls1); ss = (ss0, ss1)
        cid = lax.axis_index("c")
        sid = lax.axis_index("s")
        w = cid * _NS + sid
        zv = jnp.zeros((16,), _f32)
        ov = jnp.ones((16,), _f32)
        for r in range(_PC):
            for k in range(_D // 16):
                zz[r, pl.ds(16 * k, 16)] = zv
            for k in range(_DOUT // 16):
                zb0[r, pl.ds(16 * k, 16)] = zv
            c00[r, pl.ds(0, 16)] = zv
            ones[r, pl.ds(0, 16)] = ov

        @pl.when(sid < _G // _PC)
        def _init():
            rb = sid * _PC
            pltpu.sync_copy(zb0, pzacc.at[pl.ds(rb, _PC)])
            pltpu.sync_copy(zz, phacc.at[pl.ds(rb, _PC)])
            pltpu.sync_copy(c00, gcacc.at[pl.ds(rb, _PC)])

        plsc.subcore_barrier()

        def start_loads(k, b):
            o = (w + _W * k) * _PC
            pltpu.async_copy(agg3.at[0, pl.ds(o, _PC)], a0[b], ls[b])
            pltpu.async_copy(agg3.at[1, pl.ds(o, _PC)], a1[b], ls[b])
            pltpu.async_copy(cnt.at[0, pl.ds(o, _PC)], c0[b], ls[b])
            pltpu.async_copy(cnt.at[1, pl.ds(o, _PC)], c1[b], ls[b])
            pltpu.async_copy(h2.at[pl.ds(o, _PC)], hb[b], ls[b])
            pltpu.async_copy(batch.at[pl.ds(o, _PC)], bb[b], ls[b])

        def drain_loads(b):
            pltpu.make_async_copy(agg3.at[0, pl.ds(0, _PC)], a0[b],
                                  ls[b]).wait()
            pltpu.make_async_copy(agg3.at[1, pl.ds(0, _PC)], a1[b],
                                  ls[b]).wait()
            pltpu.make_async_copy(cnt.at[0, pl.ds(0, _PC)], c0[b],
                                  ls[b]).wait()
            pltpu.make_async_copy(cnt.at[1, pl.ds(0, _PC)], c1[b],
                                  ls[b]).wait()
            pltpu.make_async_copy(h2.at[pl.ds(0, _PC)], hb[b], ls[b]).wait()
            pltpu.make_async_copy(batch.at[pl.ds(0, _PC)], bb[b],
                                  ls[b]).wait()

        def fire_scatters(b):
            pltpu.async_copy(zb[b], pzacc.at[bb[b]], ss[b], add=True)
            pltpu.async_copy(hb[b], phacc.at[bb[b]], ss[b], add=True)
            pltpu.async_copy(ones, gcacc.at[bb[b]], ss[b], add=True)

        def drain_scatters(b):
            pltpu.make_async_copy(zb[b], pzacc.at[bb[b]], ss[b]).wait()
            pltpu.make_async_copy(hb[b], phacc.at[bb[b]], ss[b]).wait()
            pltpu.make_async_copy(ones, gcacc.at[bb[b]], ss[b]).wait()

        def compute(b):
            for r in range(_PC):
                cv = jnp.maximum(
                    c0[b][r, pl.ds(0, 16)] + c1[b][r, pl.ds(0, 16)], 1.0)
                for k in range(_DOUT // 16):
                    s = pl.ds(16 * k, 16)
                    zb[b][r, s] = (a0[b][r, s] + a1[b][r, s]) / cv

        def valid(k):
            return w + _W * k < _NCHK

        start_loads(0, 0)

        @pl.loop(0, _ITER // 2)
        def _pair(p):
            for b in range(2):
                k = 2 * p + b

                @pl.when((k >= 1) & valid(k - 1))
                def _ds():
                    drain_scatters(1 - b)

                @pl.when(valid(k + 1))
                def _pf():
                    start_loads(k + 1, 1 - b)

                @pl.when(valid(k))
                def _go():
                    drain_loads(b)
                    compute(b)
                    fire_scatters(b)

        @pl.when(valid(_ITER - 1))
        def _d1():
            drain_scatters((_ITER - 1) % 2)

        plsc.subcore_barrier()

        @pl.when(sid == 0)
        def _o0():
            pltpu.sync_copy(pzacc, pz_out.at[cid])

        @pl.when(sid == 1)
        def _o1():
            pltpu.sync_copy(phacc, ph_out.at[cid])

        @pl.when(sid == 2)
        def _o2():
            pltpu.sync_copy(gcacc, gc_out.at[cid])

    return pl.kernel(body, out_type=outs, mesh=_mesh(),
                     scratch_types=scratch,
                     compiler_params=pltpu.CompilerParams(
                         use_tc_tiling_on_sc=False))


_sc_pool = _make_sc_pool()


_R = 1000


def _dense_body(agg, cnt, xin, wl, b, wr, *rest, relu, with_y):
    if with_y:
        w3, h_out, y_out = rest
    else:
        (h_out,) = rest
    a = agg[0] + agg[1]
    c = cnt[0, :, 0:1] + cnt[1, :, 0:1]
    mean = a * (1.0 / jnp.maximum(c, 1.0))
    h = lax.dot_general(mean, wl[...], (((1,), (1,)), ((), ())),
                        preferred_element_type=_f32)
    h = h + b[...] + lax.dot_general(xin[...], wr[...], (((1,), (1,)), ((), ())),
                                     preferred_element_type=_f32)
    if relu:
        h = jnp.maximum(h, 0.0)
    h_out[...] = h
    if with_y:
        y_out[...] = lax.dot_general(h, w3[...], (((1,), (1,)), ((), ())),
                                     preferred_element_type=_f32)


def _make_dense(relu, with_y):
    in_specs = [
        pl.BlockSpec((_NC, _R, _D), lambda i: (0, i, 0)),
        pl.BlockSpec((_NC, _R, 16), lambda i: (0, i, 0)),
        pl.BlockSpec((_R, _D), lambda i: (i, 0)),
        pl.BlockSpec((_D, _D), lambda i: (0, 0)),
        pl.BlockSpec((1, _D), lambda i: (0, 0)),
        pl.BlockSpec((_D, _D), lambda i: (0, 0)),
    ]
    out_shape = [jax.ShapeDtypeStruct((_N, _D), _f32)]
    out_specs = [pl.BlockSpec((_R, _D), lambda i: (i, 0))]
    if with_y:
        in_specs.append(pl.BlockSpec((_DOUT, _D), lambda i: (0, 0)))
        out_shape.append(jax.ShapeDtypeStruct((_N, _DOUT), _f32))
        out_specs.append(pl.BlockSpec((_R, _DOUT), lambda i: (i, 0)))
    return pl.pallas_call(
        functools.partial(_dense_body, relu=relu, with_y=with_y),
        grid=(_N // _R,),
        in_specs=in_specs,
        out_specs=out_specs if with_y else out_specs[0],
        out_shape=out_shape if with_y else out_shape[0],
    )


_dense1 = _make_dense(True, False)
_dense2 = _make_dense(True, True)


def _final_body(pz, ph, gc, wr, b, out):
    z = pz[0] + pz[1]
    h = ph[0] + ph[1]
    g = gc[0, :, 0:1] + gc[1, :, 0:1]
    s = z + g * b[...] + lax.dot_general(h, wr[...], (((1,), (1,)), ((), ())),
                                         preferred_element_type=_f32)
    out[...] = s / jnp.maximum(g, 1.0)


_final = pl.pallas_call(
    _final_body,
    out_shape=jax.ShapeDtypeStruct((_G, _DOUT), _f32),
)


def kernel(x, edge_index, batch, W1l, b1, W1r, W2l, b2, W2r, W3l, b3, W3r):
    src = edge_index[0]
    dst = edge_index[1]
    cnt = _cntk(dst)
    agg1 = _agg128(x, src, dst)
    h1 = _dense1(agg1, cnt, x, W1l, b1.reshape(1, -1), W1r)
    agg2 = _agg128(h1, src, dst)
    h2, y3l = _dense2(agg2, cnt, h1, W2l, b2.reshape(1, -1), W2r, W3l)
    agg3 = _agg64(y3l, src, dst)
    pz, ph, gc = _sc_pool(agg3, cnt, h2, batch)
    return _final(pz, ph, gc, W3r, b3.reshape(1, -1))


# speedup vs baseline: 13.1632x; 1.0666x over previous
import functools

import jax
import jax.numpy as jnp
from jax import lax
from jax.experimental import pallas as pl
from jax.experimental.pallas import tpu as pltpu
from jax.experimental.pallas import tpu_sc as plsc

_N = 10000
_E = 320000
_D = 128
_DOUT = 64
_G = 128
_NC = 2
_NS = 16
_CH = 64
_EPT = _E // (_NC * _NS)
_CPT = _EPT // _CH
_TAIL = _EPT - _CPT * _CH
_SLAB = 640
_LSLAB = _N - 15 * _SLAB
_ZR = 8

_f32 = jnp.float32


def _mesh():
    return plsc.VectorSubcoreMesh(core_axis_name="c", subcore_axis_name="s")


def _make_sc_agg(D, ch, nbuf, with_cnt=False):
    cpt = _EPT // ch
    tail = _EPT - cpt * ch
    assert cpt % nbuf == 0
    outs = [jax.ShapeDtypeStruct((_NC, _N, D), _f32)]
    if with_cnt:
        outs.append(jax.ShapeDtypeStruct((_NC, _N, 16), _f32))
    scratch = (
        [pltpu.VMEM((_EPT,), jnp.int32)] * 2 +
        [pltpu.VMEM((ch, D), _f32)] * nbuf +
        [
            pltpu.VMEM((_ZR, D), _f32),
            pltpu.VMEM_SHARED((_N, D), _f32),
        ] +
        [pltpu.SemaphoreType.DMA] * nbuf +
        [pltpu.SemaphoreType.DMA] * nbuf +
        [pltpu.SemaphoreType.DMA]
    )
    if with_cnt:
        scratch += [
            pltpu.VMEM((ch, 16), _f32),
            pltpu.VMEM((_ZR, 16), _f32),
            pltpu.VMEM_SHARED((_N, 16), _f32),
            pltpu.SemaphoreType.DMA,
        ]

    def body(table, src, dst, *rest):
        if with_cnt:
            out, cnt_out = rest[:2]
            rest = rest[2:]
        else:
            out = rest[0]
            rest = rest[1:]
        srcall, dstall = rest[:2]
        gb = rest[2:2 + nbuf]
        zbuf, acc = rest[2 + nbuf:4 + nbuf]
        gs = rest[4 + nbuf:4 + 2 * nbuf]
        ss = rest[4 + 2 * nbuf:4 + 3 * nbuf]
        zs = rest[4 + 3 * nbuf]
        if with_cnt:
            ones, z16, cacc, cs = rest[5 + 3 * nbuf:]
        cid = lax.axis_index("c")
        sid = lax.axis_index("s")
        zv = jnp.zeros((16,), _f32)

        e0 = (cid * _NS + sid) * _EPT
        pltpu.async_copy(src.at[pl.ds(e0, _EPT)], srcall, gs[0])
        pltpu.async_copy(dst.at[pl.ds(e0, _EPT)], dstall, gs[1 % nbuf])

        @pl.loop(0, _ZR)
        def _zero(r):
            for k in range(D // 16):
                zbuf[r, pl.ds(16 * k, 16)] = zv
            if with_cnt:
                z16[r, pl.ds(0, 16)] = zv

        if with_cnt:
            ov = jnp.ones((16,), _f32)

            @pl.loop(0, ch)
            def _fill(r):
                ones[r, pl.ds(0, 16)] = ov

        nb = sid * _SLAB

        @pl.when(sid < _NS - 1)
        def _z_main():
            for b in range(_SLAB // _ZR):
                pltpu.async_copy(zbuf, acc.at[pl.ds(nb + b * _ZR, _ZR)], zs)
                if with_cnt:
                    pltpu.async_copy(z16, cacc.at[pl.ds(nb + b * _ZR, _ZR)],
                                     zs)
            for b in range(_SLAB // _ZR):
                pltpu.make_async_copy(zbuf, acc.at[pl.ds(nb, _ZR)], zs).wait()
                if with_cnt:
                    pltpu.make_async_copy(z16, cacc.at[pl.ds(nb, _ZR)],
                                          zs).wait()

        @pl.when(sid == _NS - 1)
        def _z_last():
            for b in range(_LSLAB // _ZR):
                o = 15 * _SLAB + b * _ZR
                pltpu.async_copy(zbuf, acc.at[pl.ds(o, _ZR)], zs)
                if with_cnt:
                    pltpu.async_copy(z16, cacc.at[pl.ds(o, _ZR)], zs)
            for b in range(_LSLAB // _ZR):
                pltpu.make_async_copy(zbuf, acc.at[pl.ds(nb, _ZR)], zs).wait()
                if with_cnt:
                    pltpu.make_async_copy(z16, cacc.at[pl.ds(nb, _ZR)],
                                          zs).wait()

        pltpu.make_async_copy(src.at[pl.ds(e0, _EPT)], srcall, gs[0]).wait()
        pltpu.make_async_copy(dst.at[pl.ds(e0, _EPT)], dstall,
                              gs[1 % nbuf]).wait()
        plsc.subcore_barrier()

        def start_gather(j, b):
            pltpu.async_copy(table.at[srcall.at[pl.ds(j * ch, ch)]],
                             gb[b], gs[b])

        def wait_gather(b):
            pltpu.make_async_copy(
                table.at[srcall.at[pl.ds(0, ch)]], gb[b], gs[b]).wait()

        def start_scatter(j, b):
            pltpu.async_copy(gb[b], acc.at[dstall.at[pl.ds(j * ch, ch)]],
                             ss[b], add=True)
            if with_cnt:
                pltpu.async_copy(ones, cacc.at[dstall.at[pl.ds(j * ch, ch)]],
                                 cs, add=True)

        def wait_scatter(b):
            pltpu.make_async_copy(
                gb[b], acc.at[dstall.at[pl.ds(0, ch)]], ss[b]).wait()

        for j in range(nbuf - 1):
            start_gather(j, j)

        @pl.loop(0, cpt // nbuf)
        def _rot(p):
            for b in range(nbuf):
                j = nbuf * p + b
                bp = (b - 1) % nbuf

                @pl.when(j >= 1)
                def _ws():
                    wait_scatter(bp)

                @pl.when(j + nbuf - 1 < cpt)
                def _sg():
                    start_gather(j + nbuf - 1, bp)
                wait_gather(b)
                start_scatter(j, b)

        wait_scatter((cpt - 1) % nbuf)
        if tail:
            et = cpt * ch
            pltpu.sync_copy(table.at[srcall.at[pl.ds(et, tail)]],
                            gb[0].at[pl.ds(0, tail)])
            pltpu.sync_copy(gb[0].at[pl.ds(0, tail)],
                            acc.at[dstall.at[pl.ds(et, tail)]], add=True)
            if with_cnt:
                pltpu.sync_copy(ones.at[pl.ds(0, tail)],
                                cacc.at[dstall.at[pl.ds(et, tail)]],
                                add=True)
        if with_cnt:
            @pl.loop(0, cpt)
            def _dc(i):
                pltpu.make_async_copy(
                    ones, cacc.at[dstall.at[pl.ds(0, ch)]], cs).wait()

        plsc.subcore_barrier()

        @pl.when(sid < _NS - 1)
        def _rb_main():
            pltpu.sync_copy(acc.at[pl.ds(nb, _SLAB)],
                            out.at[cid, pl.ds(nb, _SLAB)])
            if with_cnt:
                pltpu.sync_copy(cacc.at[pl.ds(nb, _SLAB)],
                                cnt_out.at[cid, pl.ds(nb, _SLAB)])

        @pl.when(sid == _NS - 1)
        def _rb_last():
            pltpu.sync_copy(acc.at[pl.ds(15 * _SLAB, _LSLAB)],
                            out.at[cid, pl.ds(15 * _SLAB, _LSLAB)])
            if with_cnt:
                pltpu.sync_copy(cacc.at[pl.ds(15 * _SLAB, _LSLAB)],
                                cnt_out.at[cid, pl.ds(15 * _SLAB, _LSLAB)])

    return pl.kernel(body, out_type=tuple(outs) if with_cnt else outs[0],
                     mesh=_mesh(), scratch_types=scratch,
                     compiler_params=pltpu.CompilerParams(
                         use_tc_tiling_on_sc=False))


_agg128_cnt = _make_sc_agg(_D, 64, 2, with_cnt=True)
_agg128 = _make_sc_agg(_D, 64, 2)
_agg64 = _make_sc_agg(_DOUT, 128, 2)


_PC = 16
_NCHK = _N // _PC
_W = _NC * _NS
_ITER = -(-_NCHK // _W)


def _make_sc_pool():
    outs = (jax.ShapeDtypeStruct((_NC, _G, _DOUT), _f32),
            jax.ShapeDtypeStruct((_NC, _G, _D), _f32),
            jax.ShapeDtypeStruct((_NC, _G, 16), _f32))
    nbuf = 2
    scratch = (
        [pltpu.VMEM((_PC, _DOUT), _f32)] * nbuf +
        [pltpu.VMEM((_PC, _DOUT), _f32)] * nbuf +
        [pltpu.VMEM((_PC, 16), _f32)] * nbuf +
        [pltpu.VMEM((_PC, 16), _f32)] * nbuf +
        [pltpu.VMEM((_PC, _D), _f32)] * nbuf +
        [pltpu.VMEM((_PC,), jnp.int32)] * nbuf +
        [pltpu.VMEM((_PC, _DOUT), _f32)] * nbuf +
        [
            pltpu.VMEM((_PC, 16), _f32),
            pltpu.VMEM((_PC, _D), _f32),
            pltpu.VMEM_SHARED((_G, _DOUT), _f32),
            pltpu.VMEM_SHARED((_G, _D), _f32),
            pltpu.VMEM_SHARED((_G, 16), _f32),
        ] +
        [pltpu.SemaphoreType.DMA] * nbuf +
        [pltpu.SemaphoreType.DMA] * nbuf
    )

    def body(agg3, cnt, h2, batch, pz_out, ph_out, gc_out,
             a00, a01, a10, a11, c00, c01, c10, c11, hb0, hb1, bb0, bb1,
             zb0, zb1, ones, zz, pzacc, phacc, gcacc, ls0, ls1, ss0, ss1):
        a0 = (a00, a01); a1 = (a10, a11); c0 = (c00, c01); c1 = (c10, c11)
        hb = (hb0, hb1); bb = (bb0, bb1); zb = (zb0, zb1)
        ls = (ls0, ls1); ss = (ss0, ss1)
        cid = lax.axis_index("c")
        sid = lax.axis_index("s")
        w = cid * _NS + sid
        zv = jnp.zeros((16,), _f32)
        ov = jnp.ones((16,), _f32)
        for r in range(_PC):
            for k in range(_D // 16):
                zz[r, pl.ds(16 * k, 16)] = zv
            for k in range(_DOUT // 16):
                zb0[r, pl.ds(16 * k, 16)] = zv
            c00[r, pl.ds(0, 16)] = zv
            ones[r, pl.ds(0, 16)] = ov

        @pl.when(sid < _G // _PC)
        def _init():
            rb = sid * _PC
            pltpu.sync_copy(zb0, pzacc.at[pl.ds(rb, _PC)])
            pltpu.sync_copy(zz, phacc.at[pl.ds(rb, _PC)])
            pltpu.sync_copy(c00, gcacc.at[pl.ds(rb, _PC)])

        plsc.subcore_barrier()

        def start_loads(k, b):
            o = (w + _W * k) * _PC
            pltpu.async_copy(agg3.at[0, pl.ds(o, _PC)], a0[b], ls[b])
            pltpu.async_copy(agg3.at[1, pl.ds(o, _PC)], a1[b], ls[b])
            pltpu.async_copy(cnt.at[0, pl.ds(o, _PC)], c0[b], ls[b])
            pltpu.async_copy(cnt.at[1, pl.ds(o, _PC)], c1[b], ls[b])
            pltpu.async_copy(h2.at[pl.ds(o, _PC)], hb[b], ls[b])
            pltpu.async_copy(batch.at[pl.ds(o, _PC)], bb[b], ls[b])

        def drain_loads(b):
            pltpu.make_async_copy(agg3.at[0, pl.ds(0, _PC)], a0[b],
                                  ls[b]).wait()
            pltpu.make_async_copy(agg3.at[1, pl.ds(0, _PC)], a1[b],
                                  ls[b]).wait()
            pltpu.make_async_copy(cnt.at[0, pl.ds(0, _PC)], c0[b],
                                  ls[b]).wait()
            pltpu.make_async_copy(cnt.at[1, pl.ds(0, _PC)], c1[b],
                                  ls[b]).wait()
            pltpu.make_async_copy(h2.at[pl.ds(0, _PC)], hb[b], ls[b]).wait()
            pltpu.make_async_copy(batch.at[pl.ds(0, _PC)], bb[b],
                                  ls[b]).wait()

        def fire_scatters(b):
            pltpu.async_copy(zb[b], pzacc.at[bb[b]], ss[b], add=True)
            pltpu.async_copy(hb[b], phacc.at[bb[b]], ss[b], add=True)
            pltpu.async_copy(ones, gcacc.at[bb[b]], ss[b], add=True)

        def drain_scatters(b):
            pltpu.make_async_copy(zb[b], pzacc.at[bb[b]], ss[b]).wait()
            pltpu.make_async_copy(hb[b], phacc.at[bb[b]], ss[b]).wait()
            pltpu.make_async_copy(ones, gcacc.at[bb[b]], ss[b]).wait()

        def compute(b):
            for r in range(_PC):
                cv = jnp.maximum(
                    c0[b][r, pl.ds(0, 16)] + c1[b][r, pl.ds(0, 16)], 1.0)
                for k in range(_DOUT // 16):
                    s = pl.ds(16 * k, 16)
                    zb[b][r, s] = (a0[b][r, s] + a1[b][r, s]) / cv

        def valid(k):
            return w + _W * k < _NCHK

        start_loads(0, 0)

        @pl.loop(0, _ITER // 2)
        def _pair(p):
            for b in range(2):
                k = 2 * p + b

                @pl.when((k >= 1) & valid(k - 1))
                def _ds():
                    drain_scatters(1 - b)

                @pl.when(valid(k + 1))
                def _pf():
                    start_loads(k + 1, 1 - b)

                @pl.when(valid(k))
                def _go():
                    drain_loads(b)
                    compute(b)
                    fire_scatters(b)

        @pl.when(valid(_ITER - 1))
        def _d1():
            drain_scatters((_ITER - 1) % 2)

        plsc.subcore_barrier()

        @pl.when(sid == 0)
        def _o0():
            pltpu.sync_copy(pzacc, pz_out.at[cid])

        @pl.when(sid == 1)
        def _o1():
            pltpu.sync_copy(phacc, ph_out.at[cid])

        @pl.when(sid == 2)
        def _o2():
            pltpu.sync_copy(gcacc, gc_out.at[cid])

    return pl.kernel(body, out_type=outs, mesh=_mesh(),
                     scratch_types=scratch,
                     compiler_params=pltpu.CompilerParams(
                         use_tc_tiling_on_sc=False))


_sc_pool = _make_sc_pool()


_R = 1000


def _dense_body(agg, cnt, xin, wl, b, wr, *rest, relu, with_y):
    if with_y:
        w3, h_out, y_out = rest
    else:
        (h_out,) = rest
    a = agg[0] + agg[1]
    c = cnt[0, :, 0:1] + cnt[1, :, 0:1]
    mean = a * (1.0 / jnp.maximum(c, 1.0))
    h = lax.dot_general(mean, wl[...], (((1,), (1,)), ((), ())),
                        preferred_element_type=_f32)
    h = h + b[...] + lax.dot_general(xin[...], wr[...], (((1,), (1,)), ((), ())),
                                     preferred_element_type=_f32)
    if relu:
        h = jnp.maximum(h, 0.0)
    h_out[...] = h
    if with_y:
        y_out[...] = lax.dot_general(h, w3[...], (((1,), (1,)), ((), ())),
                                     preferred_element_type=_f32)


def _make_dense(relu, with_y):
    in_specs = [
        pl.BlockSpec((_NC, _R, _D), lambda i: (0, i, 0)),
        pl.BlockSpec((_NC, _R, 16), lambda i: (0, i, 0)),
        pl.BlockSpec((_R, _D), lambda i: (i, 0)),
        pl.BlockSpec((_D, _D), lambda i: (0, 0)),
        pl.BlockSpec((1, _D), lambda i: (0, 0)),
        pl.BlockSpec((_D, _D), lambda i: (0, 0)),
    ]
    out_shape = [jax.ShapeDtypeStruct((_N, _D), _f32)]
    out_specs = [pl.BlockSpec((_R, _D), lambda i: (i, 0))]
    if with_y:
        in_specs.append(pl.BlockSpec((_DOUT, _D), lambda i: (0, 0)))
        out_shape.append(jax.ShapeDtypeStruct((_N, _DOUT), _f32))
        out_specs.append(pl.BlockSpec((_R, _DOUT), lambda i: (i, 0)))
    return pl.pallas_call(
        functools.partial(_dense_body, relu=relu, with_y=with_y),
        grid=(_N // _R,),
        in_specs=in_specs,
        out_specs=out_specs if with_y else out_specs[0],
        out_shape=out_shape if with_y else out_shape[0],
    )


_dense1 = _make_dense(True, False)
_dense2 = _make_dense(True, True)


def _final_body(pz, ph, gc, wr, b, out):
    z = pz[0] + pz[1]
    h = ph[0] + ph[1]
    g = gc[0, :, 0:1] + gc[1, :, 0:1]
    s = z + g * b[...] + lax.dot_general(h, wr[...], (((1,), (1,)), ((), ())),
                                         preferred_element_type=_f32)
    out[...] = s / jnp.maximum(g, 1.0)


_final = pl.pallas_call(
    _final_body,
    out_shape=jax.ShapeDtypeStruct((_G, _DOUT), _f32),
)


def kernel(x, edge_index, batch, W1l, b1, W1r, W2l, b2, W2r, W3l, b3, W3r):
    src = edge_index[0]
    dst = edge_index[1]
    agg1, cnt = _agg128_cnt(x, src, dst)
    h1 = _dense1(agg1, cnt, x, W1l, b1.reshape(1, -1), W1r)
    agg2 = _agg128(h1, src, dst)
    h2, y3l = _dense2(agg2, cnt, h1, W2l, b2.reshape(1, -1), W2r, W3l)
    agg3 = _agg64(y3l, src, dst)
    pz, ph, gc = _sc_pool(agg3, cnt, h2, batch)
    return _final(pz, ph, gc, W3r, b3.reshape(1, -1))


# speedup vs baseline: 14.9364x; 1.1347x over previous
import functools

import jax
import jax.numpy as jnp
from jax import lax
from jax.experimental import pallas as pl
from jax.experimental.pallas import tpu as pltpu
from jax.experimental.pallas import tpu_sc as plsc

_N = 10000
_E = 320000
_D = 128
_DOUT = 64
_G = 128
_NC = 2
_NS = 16
_CH = 64
_EPT = _E // (_NC * _NS)
_CPT = _EPT // _CH
_TAIL = _EPT - _CPT * _CH
_SLAB = 640
_LSLAB = _N - 15 * _SLAB
_ZR = 8

_f32 = jnp.float32


def _mesh():
    return plsc.VectorSubcoreMesh(core_axis_name="c", subcore_axis_name="s")


def _make_sc_agg(D, ch, nbuf, with_cnt=False):
    cpt = _EPT // ch
    tail = _EPT - cpt * ch
    assert cpt % nbuf == 0
    outs = [jax.ShapeDtypeStruct((_NC, _N, D), _f32)]
    if with_cnt:
        outs.append(jax.ShapeDtypeStruct((_NC, _N, 16), _f32))
    scratch = (
        [pltpu.VMEM((_EPT,), jnp.int32)] * 2 +
        [pltpu.VMEM((ch, D), _f32)] * nbuf +
        [
            pltpu.VMEM((_ZR, D), _f32),
            pltpu.VMEM_SHARED((_N, D), _f32),
        ] +
        [pltpu.SemaphoreType.DMA] * nbuf +
        [pltpu.SemaphoreType.DMA] * nbuf +
        [pltpu.SemaphoreType.DMA]
    )
    if with_cnt:
        scratch += [
            pltpu.VMEM((ch, 16), _f32),
            pltpu.VMEM((_ZR, 16), _f32),
            pltpu.VMEM_SHARED((_N, 16), _f32),
            pltpu.SemaphoreType.DMA,
        ]

    def body(table, src, dst, *rest):
        if with_cnt:
            out, cnt_out = rest[:2]
            rest = rest[2:]
        else:
            out = rest[0]
            rest = rest[1:]
        srcall, dstall = rest[:2]
        gb = rest[2:2 + nbuf]
        zbuf, acc = rest[2 + nbuf:4 + nbuf]
        gs = rest[4 + nbuf:4 + 2 * nbuf]
        ss = rest[4 + 2 * nbuf:4 + 3 * nbuf]
        zs = rest[4 + 3 * nbuf]
        if with_cnt:
            ones, z16, cacc, cs = rest[5 + 3 * nbuf:]
        cid = lax.axis_index("c")
        sid = lax.axis_index("s")
        zv = jnp.zeros((16,), _f32)

        e0 = (cid * _NS + sid) * _EPT
        pltpu.async_copy(src.at[pl.ds(e0, _EPT)], srcall, gs[0])
        pltpu.async_copy(dst.at[pl.ds(e0, _EPT)], dstall, gs[1 % nbuf])

        @pl.loop(0, _ZR)
        def _zero(r):
            for k in range(D // 16):
                zbuf[r, pl.ds(16 * k, 16)] = zv
            if with_cnt:
                z16[r, pl.ds(0, 16)] = zv

        if with_cnt:
            ov = jnp.ones((16,), _f32)

            @pl.loop(0, ch)
            def _fill(r):
                ones[r, pl.ds(0, 16)] = ov

        nb = sid * _SLAB

        @pl.when(sid < _NS - 1)
        def _z_main():
            for b in range(_SLAB // _ZR):
                pltpu.async_copy(zbuf, acc.at[pl.ds(nb + b * _ZR, _ZR)], zs)
                if with_cnt:
                    pltpu.async_copy(z16, cacc.at[pl.ds(nb + b * _ZR, _ZR)],
                                     zs)
            for b in range(_SLAB // _ZR):
                pltpu.make_async_copy(zbuf, acc.at[pl.ds(nb, _ZR)], zs).wait()
                if with_cnt:
                    pltpu.make_async_copy(z16, cacc.at[pl.ds(nb, _ZR)],
                                          zs).wait()

        @pl.when(sid == _NS - 1)
        def _z_last():
            for b in range(_LSLAB // _ZR):
                o = 15 * _SLAB + b * _ZR
                pltpu.async_copy(zbuf, acc.at[pl.ds(o, _ZR)], zs)
                if with_cnt:
                    pltpu.async_copy(z16, cacc.at[pl.ds(o, _ZR)], zs)
            for b in range(_LSLAB // _ZR):
                pltpu.make_async_copy(zbuf, acc.at[pl.ds(nb, _ZR)], zs).wait()
                if with_cnt:
                    pltpu.make_async_copy(z16, cacc.at[pl.ds(nb, _ZR)],
                                          zs).wait()

        pltpu.make_async_copy(src.at[pl.ds(e0, _EPT)], srcall, gs[0]).wait()
        pltpu.make_async_copy(dst.at[pl.ds(e0, _EPT)], dstall,
                              gs[1 % nbuf]).wait()
        plsc.subcore_barrier()

        def start_gather(j, b):
            pltpu.async_copy(table.at[srcall.at[pl.ds(j * ch, ch)]],
                             gb[b], gs[b])

        def wait_gather(b):
            pltpu.make_async_copy(
                table.at[srcall.at[pl.ds(0, ch)]], gb[b], gs[b]).wait()

        def start_scatter(j, b):
            pltpu.async_copy(gb[b], acc.at[dstall.at[pl.ds(j * ch, ch)]],
                             ss[b], add=True)
            if with_cnt:
                pltpu.async_copy(ones, cacc.at[dstall.at[pl.ds(j * ch, ch)]],
                                 cs, add=True)

        def wait_scatter(b):
            pltpu.make_async_copy(
                gb[b], acc.at[dstall.at[pl.ds(0, ch)]], ss[b]).wait()

        for j in range(nbuf - 1):
            start_gather(j, j)

        @pl.loop(0, cpt // nbuf)
        def _rot(p):
            for b in range(nbuf):
                j = nbuf * p + b
                bp = (b - 1) % nbuf

                @pl.when(j >= 1)
                def _ws():
                    wait_scatter(bp)

                @pl.when(j + nbuf - 1 < cpt)
                def _sg():
                    start_gather(j + nbuf - 1, bp)
                wait_gather(b)
                start_scatter(j, b)

        wait_scatter((cpt - 1) % nbuf)
        if tail:
            et = cpt * ch
            pltpu.sync_copy(table.at[srcall.at[pl.ds(et, tail)]],
                            gb[0].at[pl.ds(0, tail)])
            pltpu.sync_copy(gb[0].at[pl.ds(0, tail)],
                            acc.at[dstall.at[pl.ds(et, tail)]], add=True)
            if with_cnt:
                pltpu.sync_copy(ones.at[pl.ds(0, tail)],
                                cacc.at[dstall.at[pl.ds(et, tail)]],
                                add=True)
        if with_cnt:
            @pl.loop(0, cpt)
            def _dc(i):
                pltpu.make_async_copy(
                    ones, cacc.at[dstall.at[pl.ds(0, ch)]], cs).wait()

        plsc.subcore_barrier()

        @pl.when(sid < _NS - 1)
        def _rb_main():
            pltpu.sync_copy(acc.at[pl.ds(nb, _SLAB)],
                            out.at[cid, pl.ds(nb, _SLAB)])
            if with_cnt:
                pltpu.sync_copy(cacc.at[pl.ds(nb, _SLAB)],
                                cnt_out.at[cid, pl.ds(nb, _SLAB)])

        @pl.when(sid == _NS - 1)
        def _rb_last():
            pltpu.sync_copy(acc.at[pl.ds(15 * _SLAB, _LSLAB)],
                            out.at[cid, pl.ds(15 * _SLAB, _LSLAB)])
            if with_cnt:
                pltpu.sync_copy(cacc.at[pl.ds(15 * _SLAB, _LSLAB)],
                                cnt_out.at[cid, pl.ds(15 * _SLAB, _LSLAB)])

    return pl.kernel(body, out_type=tuple(outs) if with_cnt else outs[0],
                     mesh=_mesh(), scratch_types=scratch,
                     compiler_params=pltpu.CompilerParams(
                         use_tc_tiling_on_sc=False))


_agg128_cnt = _make_sc_agg(_D, 64, 2, with_cnt=True)
_agg128 = _make_sc_agg(_D, 40, 5)
_agg64 = _make_sc_agg(_DOUT, 128, 3)


_PC = 16
_NCHK = _N // _PC
_W = _NC * _NS
_ITER = -(-_NCHK // _W)


def _make_sc_pool():
    outs = (jax.ShapeDtypeStruct((_NC, _G, _DOUT), _f32),
            jax.ShapeDtypeStruct((_NC, _G, _D), _f32),
            jax.ShapeDtypeStruct((_NC, _G, 16), _f32))
    nbuf = 2
    scratch = (
        [pltpu.VMEM((_PC, _DOUT), _f32)] * nbuf +
        [pltpu.VMEM((_PC, _DOUT), _f32)] * nbuf +
        [pltpu.VMEM((_PC, 16), _f32)] * nbuf +
        [pltpu.VMEM((_PC, 16), _f32)] * nbuf +
        [pltpu.VMEM((_PC, _D), _f32)] * nbuf +
        [pltpu.VMEM((_PC,), jnp.int32)] * nbuf +
        [pltpu.VMEM((_PC, _DOUT), _f32)] * nbuf +
        [
            pltpu.VMEM((_PC, 16), _f32),
            pltpu.VMEM((_PC, _D), _f32),
            pltpu.VMEM_SHARED((_G, _DOUT), _f32),
            pltpu.VMEM_SHARED((_G, _D), _f32),
            pltpu.VMEM_SHARED((_G, 16), _f32),
        ] +
        [pltpu.SemaphoreType.DMA] * nbuf +
        [pltpu.SemaphoreType.DMA] * nbuf
    )

    def body(agg3, cnt, h2, batch, pz_out, ph_out, gc_out,
             a00, a01, a10, a11, c00, c01, c10, c11, hb0, hb1, bb0, bb1,
             zb0, zb1, ones, zz, pzacc, phacc, gcacc, ls0, ls1, ss0, ss1):
        a0 = (a00, a01); a1 = (a10, a11); c0 = (c00, c01); c1 = (c10, c11)
        hb = (hb0, hb1); bb = (bb0, bb1); zb = (zb0, zb1)
        ls = (ls0, ls1); ss = (ss0, ss1)
        cid = lax.axis_index("c")
        sid = lax.axis_index("s")
        w = cid * _NS + sid
        zv = jnp.zeros((16,), _f32)
        ov = jnp.ones((16,), _f32)
        for r in range(_PC):
            for k in range(_D // 16):
                zz[r, pl.ds(16 * k, 16)] = zv
            for k in range(_DOUT // 16):
                zb0[r, pl.ds(16 * k, 16)] = zv
            c00[r, pl.ds(0, 16)] = zv
            ones[r, pl.ds(0, 16)] = ov

        @pl.when(sid < _G // _PC)
        def _init():
            rb = sid * _PC
            pltpu.sync_copy(zb0, pzacc.at[pl.ds(rb, _PC)])
            pltpu.sync_copy(zz, phacc.at[pl.ds(rb, _PC)])
            pltpu.sync_copy(c00, gcacc.at[pl.ds(rb, _PC)])

        plsc.subcore_barrier()

        def start_loads(k, b):
            o = (w + _W * k) * _PC
            pltpu.async_copy(agg3.at[0, pl.ds(o, _PC)], a0[b], ls[b])
            pltpu.async_copy(agg3.at[1, pl.ds(o, _PC)], a1[b], ls[b])
            pltpu.async_copy(cnt.at[0, pl.ds(o, _PC)], c0[b], ls[b])
            pltpu.async_copy(cnt.at[1, pl.ds(o, _PC)], c1[b], ls[b])
            pltpu.async_copy(h2.at[pl.ds(o, _PC)], hb[b], ls[b])
            pltpu.async_copy(batch.at[pl.ds(o, _PC)], bb[b], ls[b])

        def drain_loads(b):
            pltpu.make_async_copy(agg3.at[0, pl.ds(0, _PC)], a0[b],
                                  ls[b]).wait()
            pltpu.make_async_copy(agg3.at[1, pl.ds(0, _PC)], a1[b],
                                  ls[b]).wait()
            pltpu.make_async_copy(cnt.at[0, pl.ds(0, _PC)], c0[b],
                                  ls[b]).wait()
            pltpu.make_async_copy(cnt.at[1, pl.ds(0, _PC)], c1[b],
                                  ls[b]).wait()
            pltpu.make_async_copy(h2.at[pl.ds(0, _PC)], hb[b], ls[b]).wait()
            pltpu.make_async_copy(batch.at[pl.ds(0, _PC)], bb[b],
                                  ls[b]).wait()

        def fire_scatters(b):
            pltpu.async_copy(zb[b], pzacc.at[bb[b]], ss[b], add=True)
            pltpu.async_copy(hb[b], phacc.at[bb[b]], ss[b], add=True)
            pltpu.async_copy(ones, gcacc.at[bb[b]], ss[b], add=True)

        def drain_scatters(b):
            pltpu.make_async_copy(zb[b], pzacc.at[bb[b]], ss[b]).wait()
            pltpu.make_async_copy(hb[b], phacc.at[bb[b]], ss[b]).wait()
            pltpu.make_async_copy(ones, gcacc.at[bb[b]], ss[b]).wait()

        def compute(b):
            for r in range(_PC):
                cv = jnp.maximum(
                    c0[b][r, pl.ds(0, 16)] + c1[b][r, pl.ds(0, 16)], 1.0)
                for k in range(_DOUT // 16):
                    s = pl.ds(16 * k, 16)
                    zb[b][r, s] = (a0[b][r, s] + a1[b][r, s]) / cv

        def valid(k):
            return w + _W * k < _NCHK

        start_loads(0, 0)

        @pl.loop(0, _ITER // 2)
        def _pair(p):
            for b in range(2):
                k = 2 * p + b

                @pl.when((k >= 1) & valid(k - 1))
                def _ds():
                    drain_scatters(1 - b)

                @pl.when(valid(k + 1))
                def _pf():
                    start_loads(k + 1, 1 - b)

                @pl.when(valid(k))
                def _go():
                    drain_loads(b)
                    compute(b)
                    fire_scatters(b)

        @pl.when(valid(_ITER - 1))
        def _d1():
            drain_scatters((_ITER - 1) % 2)

        plsc.subcore_barrier()

        @pl.when(sid == 0)
        def _o0():
            pltpu.sync_copy(pzacc, pz_out.at[cid])

        @pl.when(sid == 1)
        def _o1():
            pltpu.sync_copy(phacc, ph_out.at[cid])

        @pl.when(sid == 2)
        def _o2():
            pltpu.sync_copy(gcacc, gc_out.at[cid])

    return pl.kernel(body, out_type=outs, mesh=_mesh(),
                     scratch_types=scratch,
                     compiler_params=pltpu.CompilerParams(
                         use_tc_tiling_on_sc=False))


_sc_pool = _make_sc_pool()


_R = 1000


def _dense_body(agg, cnt, xin, wl, b, wr, *rest, relu, with_y):
    if with_y:
        w3, h_out, y_out = rest
    else:
        (h_out,) = rest
    a = agg[0] + agg[1]
    c = cnt[0, :, 0:1] + cnt[1, :, 0:1]
    mean = a * (1.0 / jnp.maximum(c, 1.0))
    h = lax.dot_general(mean, wl[...], (((1,), (1,)), ((), ())),
                        preferred_element_type=_f32)
    h = h + b[...] + lax.dot_general(xin[...], wr[...], (((1,), (1,)), ((), ())),
                                     preferred_element_type=_f32)
    if relu:
        h = jnp.maximum(h, 0.0)
    h_out[...] = h
    if with_y:
        y_out[...] = lax.dot_general(h, w3[...], (((1,), (1,)), ((), ())),
                                     preferred_element_type=_f32)


def _make_dense(relu, with_y):
    in_specs = [
        pl.BlockSpec((_NC, _R, _D), lambda i: (0, i, 0)),
        pl.BlockSpec((_NC, _R, 16), lambda i: (0, i, 0)),
        pl.BlockSpec((_R, _D), lambda i: (i, 0)),
        pl.BlockSpec((_D, _D), lambda i: (0, 0)),
        pl.BlockSpec((1, _D), lambda i: (0, 0)),
        pl.BlockSpec((_D, _D), lambda i: (0, 0)),
    ]
    out_shape = [jax.ShapeDtypeStruct((_N, _D), _f32)]
    out_specs = [pl.BlockSpec((_R, _D), lambda i: (i, 0))]
    if with_y:
        in_specs.append(pl.BlockSpec((_DOUT, _D), lambda i: (0, 0)))
        out_shape.append(jax.ShapeDtypeStruct((_N, _DOUT), _f32))
        out_specs.append(pl.BlockSpec((_R, _DOUT), lambda i: (i, 0)))
    return pl.pallas_call(
        functools.partial(_dense_body, relu=relu, with_y=with_y),
        grid=(_N // _R,),
        in_specs=in_specs,
        out_specs=out_specs if with_y else out_specs[0],
        out_shape=out_shape if with_y else out_shape[0],
    )


_dense1 = _make_dense(True, False)
_dense2 = _make_dense(True, True)


def _final_body(pz, ph, gc, wr, b, out):
    z = pz[0] + pz[1]
    h = ph[0] + ph[1]
    g = gc[0, :, 0:1] + gc[1, :, 0:1]
    s = z + g * b[...] + lax.dot_general(h, wr[...], (((1,), (1,)), ((), ())),
                                         preferred_element_type=_f32)
    out[...] = s / jnp.maximum(g, 1.0)


_final = pl.pallas_call(
    _final_body,
    out_shape=jax.ShapeDtypeStruct((_G, _DOUT), _f32),
)


def kernel(x, edge_index, batch, W1l, b1, W1r, W2l, b2, W2r, W3l, b3, W3r):
    src = edge_index[0]
    dst = edge_index[1]
    agg1, cnt = _agg128_cnt(x, src, dst)
    h1 = _dense1(agg1, cnt, x, W1l, b1.reshape(1, -1), W1r)
    agg2 = _agg128(h1, src, dst)
    h2, y3l = _dense2(agg2, cnt, h1, W2l, b2.reshape(1, -1), W2r, W3l)
    agg3 = _agg64(y3l, src, dst)
    pz, ph, gc = _sc_pool(agg3, cnt, h2, batch)
    return _final(pz, ph, gc, W3r, b3.reshape(1, -1))


# speedup vs baseline: 16.0013x; 1.0713x over previous
import functools

import jax
import jax.numpy as jnp
from jax import lax
from jax.experimental import pallas as pl
from jax.experimental.pallas import tpu as pltpu
from jax.experimental.pallas import tpu_sc as plsc

_N = 10000
_E = 320000
_D = 128
_DOUT = 64
_G = 128
_NC = 2
_NS = 16
_CH = 64
_EPT = _E // (_NC * _NS)
_CPT = _EPT // _CH
_TAIL = _EPT - _CPT * _CH
_SLAB = 640
_LSLAB = _N - 15 * _SLAB
_ZR = 8

_f32 = jnp.float32


def _mesh():
    return plsc.VectorSubcoreMesh(core_axis_name="c", subcore_axis_name="s")


def _make_sc_agg(D, ch, nbuf, with_cnt=False):
    cpt = _EPT // ch
    tail = _EPT - cpt * ch
    assert cpt % nbuf == 0
    outs = [jax.ShapeDtypeStruct((_NC, _N, D), _f32)]
    if with_cnt:
        outs.append(jax.ShapeDtypeStruct((_NC, _N, 16), _f32))
    scratch = (
        [pltpu.VMEM((_EPT,), jnp.int32)] * 2 +
        [pltpu.VMEM((ch, D), _f32)] * nbuf +
        [
            pltpu.VMEM((_ZR, D), _f32),
            pltpu.VMEM_SHARED((_N, D), _f32),
        ] +
        [pltpu.SemaphoreType.DMA] * nbuf +
        [pltpu.SemaphoreType.DMA] * nbuf +
        [pltpu.SemaphoreType.DMA]
    )
    if with_cnt:
        scratch += [
            pltpu.VMEM((ch, 16), _f32),
            pltpu.VMEM((_ZR, 16), _f32),
            pltpu.VMEM_SHARED((_N, 16), _f32),
            pltpu.SemaphoreType.DMA,
        ]

    def body(table, src, dst, *rest):
        if with_cnt:
            out, cnt_out = rest[:2]
            rest = rest[2:]
        else:
            out = rest[0]
            rest = rest[1:]
        srcall, dstall = rest[:2]
        gb = rest[2:2 + nbuf]
        zbuf, acc = rest[2 + nbuf:4 + nbuf]
        gs = rest[4 + nbuf:4 + 2 * nbuf]
        ss = rest[4 + 2 * nbuf:4 + 3 * nbuf]
        zs = rest[4 + 3 * nbuf]
        if with_cnt:
            ones, z16, cacc, cs = rest[5 + 3 * nbuf:]
        cid = lax.axis_index("c")
        sid = lax.axis_index("s")
        zv = jnp.zeros((16,), _f32)

        e0 = (cid * _NS + sid) * _EPT
        pltpu.async_copy(src.at[pl.ds(e0, _EPT)], srcall, gs[0])
        pltpu.async_copy(dst.at[pl.ds(e0, _EPT)], dstall, gs[1 % nbuf])

        @pl.loop(0, _ZR)
        def _zero(r):
            for k in range(D // 16):
                zbuf[r, pl.ds(16 * k, 16)] = zv
            if with_cnt:
                z16[r, pl.ds(0, 16)] = zv

        if with_cnt:
            ov = jnp.ones((16,), _f32)

            @pl.loop(0, ch)
            def _fill(r):
                ones[r, pl.ds(0, 16)] = ov

        nb = sid * _SLAB

        @pl.when(sid < _NS - 1)
        def _z_main():
            for b in range(_SLAB // _ZR):
                pltpu.async_copy(zbuf, acc.at[pl.ds(nb + b * _ZR, _ZR)], zs)
                if with_cnt:
                    pltpu.async_copy(z16, cacc.at[pl.ds(nb + b * _ZR, _ZR)],
                                     zs)
            for b in range(_SLAB // _ZR):
                pltpu.make_async_copy(zbuf, acc.at[pl.ds(nb, _ZR)], zs).wait()
                if with_cnt:
                    pltpu.make_async_copy(z16, cacc.at[pl.ds(nb, _ZR)],
                                          zs).wait()

        @pl.when(sid == _NS - 1)
        def _z_last():
            for b in range(_LSLAB // _ZR):
                o = 15 * _SLAB + b * _ZR
                pltpu.async_copy(zbuf, acc.at[pl.ds(o, _ZR)], zs)
                if with_cnt:
                    pltpu.async_copy(z16, cacc.at[pl.ds(o, _ZR)], zs)
            for b in range(_LSLAB // _ZR):
                pltpu.make_async_copy(zbuf, acc.at[pl.ds(nb, _ZR)], zs).wait()
                if with_cnt:
                    pltpu.make_async_copy(z16, cacc.at[pl.ds(nb, _ZR)],
                                          zs).wait()

        pltpu.make_async_copy(src.at[pl.ds(e0, _EPT)], srcall, gs[0]).wait()
        pltpu.make_async_copy(dst.at[pl.ds(e0, _EPT)], dstall,
                              gs[1 % nbuf]).wait()
        plsc.subcore_barrier()

        def start_gather(j, b):
            pltpu.async_copy(table.at[srcall.at[pl.ds(j * ch, ch)]],
                             gb[b], gs[b])

        def wait_gather(b):
            pltpu.make_async_copy(
                table.at[srcall.at[pl.ds(0, ch)]], gb[b], gs[b]).wait()

        def start_scatter(j, b):
            pltpu.async_copy(gb[b], acc.at[dstall.at[pl.ds(j * ch, ch)]],
                             ss[b], add=True)
            if with_cnt:
                pltpu.async_copy(ones, cacc.at[dstall.at[pl.ds(j * ch, ch)]],
                                 cs, add=True)

        def wait_scatter(b):
            pltpu.make_async_copy(
                gb[b], acc.at[dstall.at[pl.ds(0, ch)]], ss[b]).wait()

        for j in range(nbuf - 1):
            start_gather(j, j)

        @pl.loop(0, cpt // nbuf)
        def _rot(p):
            for b in range(nbuf):
                j = nbuf * p + b
                bp = (b - 1) % nbuf

                @pl.when(j >= 1)
                def _ws():
                    wait_scatter(bp)

                @pl.when(j + nbuf - 1 < cpt)
                def _sg():
                    start_gather(j + nbuf - 1, bp)
                wait_gather(b)
                start_scatter(j, b)

        wait_scatter((cpt - 1) % nbuf)
        if tail:
            et = cpt * ch
            pltpu.sync_copy(table.at[srcall.at[pl.ds(et, tail)]],
                            gb[0].at[pl.ds(0, tail)])
            pltpu.sync_copy(gb[0].at[pl.ds(0, tail)],
                            acc.at[dstall.at[pl.ds(et, tail)]], add=True)
            if with_cnt:
                pltpu.sync_copy(ones.at[pl.ds(0, tail)],
                                cacc.at[dstall.at[pl.ds(et, tail)]],
                                add=True)
        if with_cnt:
            @pl.loop(0, cpt)
            def _dc(i):
                pltpu.make_async_copy(
                    ones, cacc.at[dstall.at[pl.ds(0, ch)]], cs).wait()

        plsc.subcore_barrier()

        @pl.when(sid < _NS - 1)
        def _rb_main():
            pltpu.sync_copy(acc.at[pl.ds(nb, _SLAB)],
                            out.at[cid, pl.ds(nb, _SLAB)])
            if with_cnt:
                pltpu.sync_copy(cacc.at[pl.ds(nb, _SLAB)],
                                cnt_out.at[cid, pl.ds(nb, _SLAB)])

        @pl.when(sid == _NS - 1)
        def _rb_last():
            pltpu.sync_copy(acc.at[pl.ds(15 * _SLAB, _LSLAB)],
                            out.at[cid, pl.ds(15 * _SLAB, _LSLAB)])
            if with_cnt:
                pltpu.sync_copy(cacc.at[pl.ds(15 * _SLAB, _LSLAB)],
                                cnt_out.at[cid, pl.ds(15 * _SLAB, _LSLAB)])

    return pl.kernel(body, out_type=tuple(outs) if with_cnt else outs[0],
                     mesh=_mesh(), scratch_types=scratch,
                     compiler_params=pltpu.CompilerParams(
                         use_tc_tiling_on_sc=False))


_agg128_cnt = _make_sc_agg(_D, 32, 4, with_cnt=True)
_agg128 = _make_sc_agg(_D, 40, 5)
_agg64 = _make_sc_agg(_DOUT, 80, 5)


_PC = 16
_NCHK = _N // _PC
_W = _NC * _NS
_ITER = -(-_NCHK // _W)


def _make_sc_pool():
    outs = (jax.ShapeDtypeStruct((_NC, _G, _DOUT), _f32),
            jax.ShapeDtypeStruct((_NC, _G, _D), _f32),
            jax.ShapeDtypeStruct((_NC, _G, 16), _f32))
    nbuf = 2
    scratch = (
        [pltpu.VMEM((_PC, _DOUT), _f32)] * nbuf +
        [pltpu.VMEM((_PC, _DOUT), _f32)] * nbuf +
        [pltpu.VMEM((_PC, 16), _f32)] * nbuf +
        [pltpu.VMEM((_PC, 16), _f32)] * nbuf +
        [pltpu.VMEM((_PC, _D), _f32)] * nbuf +
        [pltpu.VMEM((_PC,), jnp.int32)] * nbuf +
        [pltpu.VMEM((_PC, _DOUT), _f32)] * nbuf +
        [
            pltpu.VMEM((_PC, 16), _f32),
            pltpu.VMEM((_PC, _D), _f32),
            pltpu.VMEM_SHARED((_G, _DOUT), _f32),
            pltpu.VMEM_SHARED((_G, _D), _f32),
            pltpu.VMEM_SHARED((_G, 16), _f32),
        ] +
        [pltpu.SemaphoreType.DMA] * nbuf +
        [pltpu.SemaphoreType.DMA] * nbuf
    )

    def body(agg3, cnt, h2, batch, pz_out, ph_out, gc_out,
             a00, a01, a10, a11, c00, c01, c10, c11, hb0, hb1, bb0, bb1,
             zb0, zb1, ones, zz, pzacc, phacc, gcacc, ls0, ls1, ss0, ss1):
        a0 = (a00, a01); a1 = (a10, a11); c0 = (c00, c01); c1 = (c10, c11)
        hb = (hb0, hb1); bb = (bb0, bb1); zb = (zb0, zb1)
        ls = (ls0, ls1); ss = (ss0, ss1)
        cid = lax.axis_index("c")
        sid = lax.axis_index("s")
        w = cid * _NS + sid
        zv = jnp.zeros((16,), _f32)
        ov = jnp.ones((16,), _f32)
        for r in range(_PC):
            for k in range(_D // 16):
                zz[r, pl.ds(16 * k, 16)] = zv
            for k in range(_DOUT // 16):
                zb0[r, pl.ds(16 * k, 16)] = zv
            c00[r, pl.ds(0, 16)] = zv
            ones[r, pl.ds(0, 16)] = ov

        @pl.when(sid < _G // _PC)
        def _init():
            rb = sid * _PC
            pltpu.sync_copy(zb0, pzacc.at[pl.ds(rb, _PC)])
            pltpu.sync_copy(zz, phacc.at[pl.ds(rb, _PC)])
            pltpu.sync_copy(c00, gcacc.at[pl.ds(rb, _PC)])

        plsc.subcore_barrier()

        def start_loads(k, b):
            o = (w + _W * k) * _PC
            pltpu.async_copy(agg3.at[0, pl.ds(o, _PC)], a0[b], ls[b])
            pltpu.async_copy(agg3.at[1, pl.ds(o, _PC)], a1[b], ls[b])
            pltpu.async_copy(cnt.at[0, pl.ds(o, _PC)], c0[b], ls[b])
            pltpu.async_copy(cnt.at[1, pl.ds(o, _PC)], c1[b], ls[b])
            pltpu.async_copy(h2.at[pl.ds(o, _PC)], hb[b], ls[b])
            pltpu.async_copy(batch.at[pl.ds(o, _PC)], bb[b], ls[b])

        def drain_loads(b):
            pltpu.make_async_copy(agg3.at[0, pl.ds(0, _PC)], a0[b],
                                  ls[b]).wait()
            pltpu.make_async_copy(agg3.at[1, pl.ds(0, _PC)], a1[b],
                                  ls[b]).wait()
            pltpu.make_async_copy(cnt.at[0, pl.ds(0, _PC)], c0[b],
                                  ls[b]).wait()
            pltpu.make_async_copy(cnt.at[1, pl.ds(0, _PC)], c1[b],
                                  ls[b]).wait()
            pltpu.make_async_copy(h2.at[pl.ds(0, _PC)], hb[b], ls[b]).wait()
            pltpu.make_async_copy(batch.at[pl.ds(0, _PC)], bb[b],
                                  ls[b]).wait()

        def fire_scatters(b):
            pltpu.async_copy(zb[b], pzacc.at[bb[b]], ss[b], add=True)
            pltpu.async_copy(hb[b], phacc.at[bb[b]], ss[b], add=True)
            pltpu.async_copy(ones, gcacc.at[bb[b]], ss[b], add=True)

        def drain_scatters(b):
            pltpu.make_async_copy(zb[b], pzacc.at[bb[b]], ss[b]).wait()
            pltpu.make_async_copy(hb[b], phacc.at[bb[b]], ss[b]).wait()
            pltpu.make_async_copy(ones, gcacc.at[bb[b]], ss[b]).wait()

        def compute(b):
            for r in range(_PC):
                cv = jnp.maximum(
                    c0[b][r, pl.ds(0, 16)] + c1[b][r, pl.ds(0, 16)], 1.0)
                for k in range(_DOUT // 16):
                    s = pl.ds(16 * k, 16)
                    zb[b][r, s] = (a0[b][r, s] + a1[b][r, s]) / cv

        def valid(k):
            return w + _W * k < _NCHK

        start_loads(0, 0)

        @pl.loop(0, _ITER // 2)
        def _pair(p):
            for b in range(2):
                k = 2 * p + b

                @pl.when((k >= 1) & valid(k - 1))
                def _ds():
                    drain_scatters(1 - b)

                @pl.when(valid(k + 1))
                def _pf():
                    start_loads(k + 1, 1 - b)

                @pl.when(valid(k))
                def _go():
                    drain_loads(b)
                    compute(b)
                    fire_scatters(b)

        @pl.when(valid(_ITER - 1))
        def _d1():
            drain_scatters((_ITER - 1) % 2)

        plsc.subcore_barrier()

        @pl.when(sid == 0)
        def _o0():
            pltpu.sync_copy(pzacc, pz_out.at[cid])

        @pl.when(sid == 1)
        def _o1():
            pltpu.sync_copy(phacc, ph_out.at[cid])

        @pl.when(sid == 2)
        def _o2():
            pltpu.sync_copy(gcacc, gc_out.at[cid])

    return pl.kernel(body, out_type=outs, mesh=_mesh(),
                     scratch_types=scratch,
                     compiler_params=pltpu.CompilerParams(
                         use_tc_tiling_on_sc=False))


_sc_pool = _make_sc_pool()


_R = 1000


def _dense_body(agg, cnt, xin, wl, b, wr, *rest, relu, with_y):
    if with_y:
        w3, h_out, y_out = rest
    else:
        (h_out,) = rest
    a = agg[0] + agg[1]
    c = cnt[0, :, 0:1] + cnt[1, :, 0:1]
    mean = a * (1.0 / jnp.maximum(c, 1.0))
    h = lax.dot_general(mean, wl[...], (((1,), (1,)), ((), ())),
                        preferred_element_type=_f32)
    h = h + b[...] + lax.dot_general(xin[...], wr[...], (((1,), (1,)), ((), ())),
                                     preferred_element_type=_f32)
    if relu:
        h = jnp.maximum(h, 0.0)
    h_out[...] = h
    if with_y:
        y_out[...] = lax.dot_general(h, w3[...], (((1,), (1,)), ((), ())),
                                     preferred_element_type=_f32)


def _make_dense(relu, with_y):
    in_specs = [
        pl.BlockSpec((_NC, _R, _D), lambda i: (0, i, 0)),
        pl.BlockSpec((_NC, _R, 16), lambda i: (0, i, 0)),
        pl.BlockSpec((_R, _D), lambda i: (i, 0)),
        pl.BlockSpec((_D, _D), lambda i: (0, 0)),
        pl.BlockSpec((1, _D), lambda i: (0, 0)),
        pl.BlockSpec((_D, _D), lambda i: (0, 0)),
    ]
    out_shape = [jax.ShapeDtypeStruct((_N, _D), _f32)]
    out_specs = [pl.BlockSpec((_R, _D), lambda i: (i, 0))]
    if with_y:
        in_specs.append(pl.BlockSpec((_DOUT, _D), lambda i: (0, 0)))
        out_shape.append(jax.ShapeDtypeStruct((_N, _DOUT), _f32))
        out_specs.append(pl.BlockSpec((_R, _DOUT), lambda i: (i, 0)))
    return pl.pallas_call(
        functools.partial(_dense_body, relu=relu, with_y=with_y),
        grid=(_N // _R,),
        in_specs=in_specs,
        out_specs=out_specs if with_y else out_specs[0],
        out_shape=out_shape if with_y else out_shape[0],
    )


_dense1 = _make_dense(True, False)
_dense2 = _make_dense(True, True)


def _final_body(pz, ph, gc, wr, b, out):
    z = pz[0] + pz[1]
    h = ph[0] + ph[1]
    g = gc[0, :, 0:1] + gc[1, :, 0:1]
    s = z + g * b[...] + lax.dot_general(h, wr[...], (((1,), (1,)), ((), ())),
                                         preferred_element_type=_f32)
    out[...] = s / jnp.maximum(g, 1.0)


_final = pl.pallas_call(
    _final_body,
    out_shape=jax.ShapeDtypeStruct((_G, _DOUT), _f32),
)


def kernel(x, edge_index, batch, W1l, b1, W1r, W2l, b2, W2r, W3l, b3, W3r):
    src = edge_index[0]
    dst = edge_index[1]
    agg1, cnt = _agg128_cnt(x, src, dst)
    h1 = _dense1(agg1, cnt, x, W1l, b1.reshape(1, -1), W1r)
    agg2 = _agg128(h1, src, dst)
    h2, y3l = _dense2(agg2, cnt, h1, W2l, b2.reshape(1, -1), W2r, W3l)
    agg3 = _agg64(y3l, src, dst)
    pz, ph, gc = _sc_pool(agg3, cnt, h2, batch)
    return _final(pz, ph, gc, W3r, b3.reshape(1, -1))
